# baseline (speedup 1.0000x reference)
import functools

import jax
import jax.numpy as jnp
from jax import lax
from jax.experimental import pallas as pl
from jax.experimental.pallas import tpu as pltpu
from jax.experimental.pallas import tpu_sc as plsc

NC, NS = 2, 16
NW = NC * NS
CH = 80


def _sc_gather2(tA, tB, idxA3, idxB3):
    _, rows_w, _ = idxA3.shape
    E = NW * rows_w * CH
    DA = tA.shape[1]
    DB = tB.shape[1]
    mesh = plsc.VectorSubcoreMesh(core_axis_name="c", subcore_axis_name="s")

    @functools.partial(
        pl.kernel,
        out_type=jax.ShapeDtypeStruct((E, DA + DB), jnp.float32),
        mesh=mesh,
        compiler_params=pltpu.CompilerParams(needs_layout_passes=False),
        scratch_types=[
            pltpu.VMEM((rows_w, CH), jnp.int32),
            pltpu.VMEM((rows_w, CH), jnp.int32),
            pltpu.VMEM((CH, DA + DB), jnp.float32),
            pltpu.VMEM((CH, DA + DB), jnp.float32),
            pltpu.SemaphoreType.DMA,
            pltpu.SemaphoreType.DMA,
            pltpu.SemaphoreType.DMA,
            pltpu.SemaphoreType.DMA,
        ],
    )
    def k(tA_h, tB_h, iA_h, iB_h, outAB,
          idxA_v, idxB_v, buf0, buf1, sA0, sA1, sB0, sB1):
        wid = lax.axis_index("s") * NC + lax.axis_index("c")
        row0 = wid * rows_w
        pltpu.sync_copy(iA_h.at[wid], idxA_v)
        pltpu.sync_copy(iB_h.at[wid], idxB_v)
        buf = (buf0, buf1)
        sA = (sA0, sA1)
        sB = (sB0, sB1)

        def issue(i, s):
            pltpu.async_copy(tA_h.at[idxA_v.at[i]],
                             buf[s].at[:, pl.ds(0, DA)], sA[s])
            pltpu.async_copy(tB_h.at[idxB_v.at[i]],
                             buf[s].at[:, pl.ds(DA, DB)], sB[s])

        def finish(i, s):
            pltpu.make_async_copy(tA_h.at[idxA_v.at[i]],
                                  buf[s].at[:, pl.ds(0, DA)], sA[s]).wait()
            pltpu.make_async_copy(tB_h.at[idxB_v.at[i]],
                                  buf[s].at[:, pl.ds(DA, DB)], sB[s]).wait()
            off = (row0 + i) * CH
            pltpu.sync_copy(buf[s], outAB.at[pl.ds(off, CH)])

        issue(0, 0)

        def body(g, _):
            for b in range(2):
                i = 2 * g + b
                @pl.when(i + 1 < rows_w)
                def _():
                    issue(i + 1, 1 - b)

                finish(i, b)
            return 0

        lax.fori_loop(0, rows_w // 2, body, 0)

    return k(tA, tB, idxA3, idxB3)


def _sc_scatter_add(m, dst3, n_rows, zslab):
    E, D = m.shape
    rows_w = E // (NW * CH)
    slab = n_rows // NS
    mesh = plsc.VectorSubcoreMesh(core_axis_name="c", subcore_axis_name="s")

    @functools.partial(
        pl.kernel,
        out_type=jax.ShapeDtypeStruct((NC, n_rows, D), jnp.float32),
        mesh=mesh,
        compiler_params=pltpu.CompilerParams(needs_layout_passes=False),
        scratch_types=[
            pltpu.VMEM((rows_w, CH), jnp.int32),
            pltpu.VMEM_SHARED((n_rows, D), jnp.float32),
        ] + [pltpu.VMEM((CH, D), jnp.float32)] * 2
          + [pltpu.SemaphoreType.DMA] * 2,
    )
    def k(m_h, dst_h, z_h, out, idx_v, acc, *bufs):
        nbuf = 2
        buf = bufs[:nbuf]
        sem = bufs[nbuf:2 * nbuf]
        cid = lax.axis_index("c")
        sid = lax.axis_index("s")
        wid = sid * NC + cid
        row0 = wid * rows_w
        pltpu.sync_copy(z_h, acc.at[pl.ds(sid * slab, slab)])
        pltpu.sync_copy(dst_h.at[wid], idx_v)
        plsc.subcore_barrier()

        def l_issue(i, s):
            pltpu.async_copy(m_h.at[pl.ds((row0 + i) * CH, CH)], buf[s], sem[s])

        def l_wait(i, s):
            pltpu.make_async_copy(
                m_h.at[pl.ds((row0 + i) * CH, CH)], buf[s], sem[s]).wait()

        l_issue(0, 0)

        def body(g, _):
            for b in range(nbuf):
                i = nbuf * g + b
                l_wait(i, b)

                @pl.when(i + 1 < rows_w)
                def _():
                    l_issue(i + 1, 1 - b)

                pltpu.sync_copy(buf[b], acc.at[idx_v.at[i]], add=True)
            return 0

        lax.fori_loop(0, rows_w // nbuf, body, 0)
        plsc.subcore_barrier()
        pltpu.sync_copy(acc.at[pl.ds(sid * slab, slab)],
                        out.at[cid, pl.ds(sid * slab, slab)])

    return k(m, dst3, zslab)


def _sc_edge_sumsq(pos4, il3, ir3):
    _, rows_w, _ = il3.shape
    E = NW * rows_w * CH
    per_w = rows_w * CH
    N4 = pos4.shape[0]
    groups = CH // 16
    mesh = plsc.VectorSubcoreMesh(core_axis_name="c", subcore_axis_name="s")

    @functools.partial(
        pl.kernel,
        out_type=jax.ShapeDtypeStruct((E,), jnp.float32),
        mesh=mesh,
        compiler_params=pltpu.CompilerParams(needs_layout_passes=False),
        scratch_types=[
            pltpu.VMEM((N4,), jnp.float32),
            pltpu.VMEM((rows_w, CH), jnp.int32),
            pltpu.VMEM((rows_w, CH), jnp.int32),
            pltpu.VMEM((per_w,), jnp.float32),
        ],
    )
    def k(pos_h, iL_h, iR_h, out, pos_v, iL_v, iR_v, out_v):
        wid = lax.axis_index("s") * NC + lax.axis_index("c")
        pltpu.sync_copy(pos_h, pos_v)
        pltpu.sync_copy(iL_h.at[wid], iL_v)
        pltpu.sync_copy(iR_h.at[wid], iR_v)

        def body(r, _):
            for g in range(groups):
                il = iL_v[r, pl.ds(g * 16, 16)] * 4
                ir = iR_v[r, pl.ds(g * 16, 16)] * 4
                acc = jnp.zeros((16,), jnp.float32)
                for c in range(3):
                    a = plsc.load_gather(pos_v, [il + c])
                    b = plsc.load_gather(pos_v, [ir + c])
                    d = a - b
                    acc = acc + d * d
                out_v[pl.ds(r * CH + g * 16, 16)] = acc
            return 0

        lax.fori_loop(0, rows_w, body, 0)
        pltpu.sync_copy(out_v, out.at[pl.ds(wid * per_w, per_w)])

    return k(pos4, il3, ir3)


def _silu(v):
    return v * jax.nn.sigmoid(v)


def _tc_embed(xg, W_emb, b_emb, WA, WB, nb):
    N, D = xg.shape
    H = W_emb.shape[1]
    grid = N // nb

    def body(x_ref, we, be, wa, wb, h_ref, a_ref, b_ref):
        h = jnp.dot(x_ref[...], we[...],
                    preferred_element_type=jnp.float32) + be[...]
        h_ref[...] = h
        a_ref[...] = jnp.dot(h, wa[...], preferred_element_type=jnp.float32)
        b_ref[...] = jnp.dot(h, wb[...], preferred_element_type=jnp.float32)

    w_spec = [pl.BlockSpec((D, H), lambda i: (0, 0)),
              pl.BlockSpec((1, H), lambda i: (0, 0)),
              pl.BlockSpec((H, H), lambda i: (0, 0)),
              pl.BlockSpec((H, H), lambda i: (0, 0))]
    return pl.pallas_call(
        body,
        grid=(grid,),
        in_specs=[pl.BlockSpec((nb, D), lambda i: (i, 0))] + w_spec,
        out_specs=[pl.BlockSpec((nb, H), lambda i: (i, 0))] * 3,
        out_shape=[jax.ShapeDtypeStruct((N, H), jnp.float32)] * 3,
    )(xg, W_emb, b_emb, WA, WB)


def _tc_message(AB, sumsq_c, w_inv, bm1, Wm2, bm2, eb):
    E = AB.shape[0]
    H = AB.shape[1] // 2
    grid = E // eb
    rb = eb // 128

    def body(ab_ref, s_ref, wi, b1, w2, b2, m_ref):
        ab = ab_ref[...]
        invc = jnp.sqrt(s_ref[...] + 1e-12)
        row = lax.broadcasted_iota(jnp.int32, (eb, rb), 0) // 128
        col = lax.broadcasted_iota(jnp.int32, (eb, rb), 1)
        sel = (row == col).astype(jnp.float32)
        t = jnp.dot(sel, invc, preferred_element_type=jnp.float32)
        lane = lax.broadcasted_iota(jnp.int32, (eb, 128), 1)
        rmod = lax.broadcasted_iota(jnp.int32, (eb, 128), 0) % 128
        inv_col = jnp.sum(jnp.where(lane == rmod, t, 0.0), axis=1,
                          keepdims=True)
        z = ab[:, :H] + ab[:, H:] + inv_col * wi[...] + b1[...]
        z = _silu(z)
        mm = jnp.dot(z, w2[...], preferred_element_type=jnp.float32) + b2[...]
        m_ref[...] = _silu(mm)

    return pl.pallas_call(
        body,
        grid=(grid,),
        in_specs=[
            pl.BlockSpec((eb, 2 * H), lambda i: (i, 0)),
            pl.BlockSpec((rb, 128), lambda i: (i, 0)),
            pl.BlockSpec((1, H), lambda i: (0, 0)),
            pl.BlockSpec((1, H), lambda i: (0, 0)),
            pl.BlockSpec((H, H), lambda i: (0, 0)),
            pl.BlockSpec((1, H), lambda i: (0, 0)),
        ],
        out_specs=pl.BlockSpec((eb, H), lambda i: (i, 0)),
        out_shape=jax.ShapeDtypeStruct((E, H), jnp.float32),
    )(AB, sumsq_c, w_inv, bm1, Wm2, bm2)


def _tc_update(h, agg0, agg1, W1h, W1a, b1, W2, b2, WA, WB, nb):
    N, H = h.shape
    grid = N // nb
    n_out = 3 if WA is not None else 1

    def body(h_ref, a0_ref, a1_ref, w1h, w1a, b1r, w2, b2r, *outs):
        h_blk = h_ref[...]
        agg = a0_ref[...] + a1_ref[...]
        u = jnp.dot(h_blk, w1h[...], preferred_element_type=jnp.float32)
        u += jnp.dot(agg, w1a[...], preferred_element_type=jnp.float32)
        u = _silu(u + b1r[...])
        u = jnp.dot(u, w2[...], preferred_element_type=jnp.float32) + b2r[...]
        hn = h_blk + u
        if n_out == 3:
            wa, wb, hn_ref, a_ref, b_ref = outs
            hn_ref[...] = hn
            a_ref[...] = jnp.dot(hn, wa[...],
                                 preferred_element_type=jnp.float32)
            b_ref[...] = jnp.dot(hn, wb[...],
                                 preferred_element_type=jnp.float32)
        else:
            (hn_ref,) = outs
            hn_ref[...] = hn

    in_specs = [
        pl.BlockSpec((nb, H), lambda i: (i, 0)),
        pl.BlockSpec((nb, H), lambda i: (i, 0)),
        pl.BlockSpec((nb, H), lambda i: (i, 0)),
        pl.BlockSpec((H, H), lambda i: (0, 0)),
        pl.BlockSpec((H, H), lambda i: (0, 0)),
        pl.BlockSpec((1, H), lambda i: (0, 0)),
        pl.BlockSpec((H, H), lambda i: (0, 0)),
        pl.BlockSpec((1, H), lambda i: (0, 0)),
    ]
    args = [h, agg0, agg1, W1h, W1a, b1, W2, b2]
    if n_out == 3:
        in_specs += [pl.BlockSpec((H, H), lambda i: (0, 0))] * 2
        args += [WA, WB]
    return pl.pallas_call(
        body,
        grid=(grid,),
        in_specs=in_specs,
        out_specs=[pl.BlockSpec((nb, H), lambda i: (i, 0))] * n_out,
        out_shape=[jax.ShapeDtypeStruct((N, H), jnp.float32)] * n_out,
    )(*args)


def _tc_pool(h, xb_row, Wp1, bp1, Wp2, bp2, Wq1, bq1, Wq2, bq2, n_graphs, nb):
    N, H = h.shape
    grid = N // nb

    def body(h_ref, xb_ref, wp1, bp1r, wp2, bp2r, wq1, bq1r, wq2, bq2r,
             out_ref, acc):
        step = pl.program_id(0)

        @pl.when(step == 0)
        def _():
            acc[...] = jnp.zeros_like(acc)

        hp = _silu(jnp.dot(h_ref[...], wp1[...],
                           preferred_element_type=jnp.float32) + bp1r[...])
        hp = jnp.dot(hp, wp2[...], preferred_element_type=jnp.float32) \
            + bp2r[...]
        rows = lax.broadcasted_iota(jnp.int32, (n_graphs, nb), 0)
        mask = (rows == xb_ref[0]).astype(jnp.float32)
        acc[...] += jnp.dot(mask, hp, preferred_element_type=jnp.float32)

        @pl.when(step == grid - 1)
        def _():
            p = acc[...]
            q = _silu(jnp.dot(p, wq1[...],
                              preferred_element_type=jnp.float32) + bq1r[...])
            out_ref[...] = jnp.dot(q, wq2[...],
                                   preferred_element_type=jnp.float32) \
                + bq2r[...]

    return pl.pallas_call(
        body,
        grid=(grid,),
        in_specs=[
            pl.BlockSpec((nb, H), lambda i: (i, 0)),
            pl.BlockSpec((1, 1, nb), lambda i: (i, 0, 0)),
            pl.BlockSpec((H, H), lambda i: (0, 0)),
            pl.BlockSpec((1, H), lambda i: (0, 0)),
            pl.BlockSpec((H, H), lambda i: (0, 0)),
            pl.BlockSpec((1, H), lambda i: (0, 0)),
            pl.BlockSpec((H, H), lambda i: (0, 0)),
            pl.BlockSpec((1, H), lambda i: (0, 0)),
            pl.BlockSpec((H, 1), lambda i: (0, 0)),
            pl.BlockSpec((1, 1), lambda i: (0, 0)),
        ],
        out_specs=pl.BlockSpec((n_graphs, 1), lambda i: (0, 0)),
        out_shape=jax.ShapeDtypeStruct((n_graphs, 1), jnp.float32),
        scratch_shapes=[pltpu.VMEM((n_graphs, H), jnp.float32)],
    )(h, xb_row, Wp1, bp1, Wp2, bp2, Wq1, bq1, Wq2, bq2)


def kernel(x, pos, feat_ind, adj, inv_ind, x_batch, W_emb, b_emb, Wm1, bm1,
           Wm2, bm2, Wu1, bu1, Wu2, bu2, Wpre1, bpre1, Wpre2, bpre2, Wpost1,
           bpost1, Wpost2, bpost2):
    N, D = x.shape
    H = W_emb.shape[1]
    L = Wm1.shape[0]
    E = adj.shape[1]
    n_graphs = 64

    xg = jnp.take(x, feat_ind, axis=0)
    NP = 10112
    EP = 327680
    pad = EP - E
    rows_w = EP // (NW * CH)
    src3 = jnp.concatenate([adj[0], jnp.zeros((pad,), jnp.int32)]
                           ).reshape(NW, rows_w, CH)
    dst3 = jnp.concatenate([adj[1], jnp.full((pad,), N, jnp.int32)]
                           ).reshape(NW, rows_w, CH)
    il3 = jnp.concatenate([inv_ind[0], jnp.zeros((pad,), jnp.int32)]
                          ).reshape(NW, rows_w, CH)
    ir3 = jnp.concatenate([inv_ind[1], jnp.zeros((pad,), jnp.int32)]
                          ).reshape(NW, rows_w, CH)
    pos4 = jnp.concatenate([pos, jnp.zeros((N, 1), jnp.float32)],
                           axis=1).reshape(-1)
    zslab = jnp.zeros((NP // NS, H), jnp.float32)

    h, A, Bt = _tc_embed(xg, W_emb, b_emb[None], Wm1[0, :H], Wm1[0, H:2 * H],
                         nb=2000)

    sumsq = _sc_edge_sumsq(pos4, il3, ir3).reshape(EP // 128, 128)

    for l in range(L):
        AB = _sc_gather2(A, Bt, src3, dst3)
        m = _tc_message(AB, sumsq, Wm1[l, 2 * H:2 * H + 1, :],
                        bm1[l][None], Wm2[l], bm2[l][None], eb=2048)
        aggp = _sc_scatter_add(m, dst3, NP, zslab)
        if l < L - 1:
            h, A, Bt = _tc_update(h, aggp[0], aggp[1], Wu1[l, :H],
                                  Wu1[l, H:], bu1[l][None], Wu2[l],
                                  bu2[l][None], Wm1[l + 1, :H],
                                  Wm1[l + 1, H:2 * H], nb=2000)
        else:
            (h,) = _tc_update(h, aggp[0], aggp[1], Wu1[l, :H], Wu1[l, H:],
                              bu1[l][None], Wu2[l], bu2[l][None],
                              None, None, nb=2000)

    xb3 = x_batch.astype(jnp.int32).reshape(N // 2000, 1, 2000)
    out = _tc_pool(h, xb3, Wpre1, bpre1[None],
                   Wpre2, bpre2[None], Wpost1, bpost1[None], Wpost2,
                   bpost2[None, :], n_graphs, nb=2000)
    return jnp.squeeze(out)


# speedup vs baseline: 1.0610x; 1.0610x over previous
import functools

import jax
import jax.numpy as jnp
from jax import lax
from jax.experimental import pallas as pl
from jax.experimental.pallas import tpu as pltpu
from jax.experimental.pallas import tpu_sc as plsc

NC, NS = 2, 16
NW = NC * NS
CH = 80


def _sc_gather2(tA, tB, idxA3, idxB3):
    _, rows_w, _ = idxA3.shape
    E = NW * rows_w * CH
    DA = tA.shape[1]
    DB = tB.shape[1]
    mesh = plsc.VectorSubcoreMesh(core_axis_name="c", subcore_axis_name="s")

    @functools.partial(
        pl.kernel,
        out_type=[jax.ShapeDtypeStruct((E, DA), jnp.float32),
                  jax.ShapeDtypeStruct((E, DB), jnp.float32)],
        mesh=mesh,
        compiler_params=pltpu.CompilerParams(needs_layout_passes=False),
        scratch_types=[
            pltpu.VMEM((rows_w, CH), jnp.int32),
            pltpu.VMEM((rows_w, CH), jnp.int32),
            pltpu.VMEM((CH, DA), jnp.float32),
            pltpu.VMEM((CH, DA), jnp.float32),
            pltpu.VMEM((CH, DB), jnp.float32),
            pltpu.VMEM((CH, DB), jnp.float32),
            pltpu.SemaphoreType.DMA,
            pltpu.SemaphoreType.DMA,
            pltpu.SemaphoreType.DMA,
            pltpu.SemaphoreType.DMA,
        ],
    )
    def k(tA_h, tB_h, iA_h, iB_h, outA, outB,
          idxA_v, idxB_v, bufA0, bufA1, bufB0, bufB1, sA0, sA1, sB0, sB1):
        wid = lax.axis_index("s") * NC + lax.axis_index("c")
        row0 = wid * rows_w
        pltpu.sync_copy(iA_h.at[wid], idxA_v)
        pltpu.sync_copy(iB_h.at[wid], idxB_v)
        bufA = (bufA0, bufA1)
        bufB = (bufB0, bufB1)
        sA = (sA0, sA1)
        sB = (sB0, sB1)

        def issue(i, s):
            pltpu.async_copy(tA_h.at[idxA_v.at[i]], bufA[s], sA[s])
            pltpu.async_copy(tB_h.at[idxB_v.at[i]], bufB[s], sB[s])

        def finish(i, s):
            off = (row0 + i) * CH
            pltpu.make_async_copy(tA_h.at[idxA_v.at[i]], bufA[s], sA[s]).wait()
            pltpu.sync_copy(bufA[s], outA.at[pl.ds(off, CH)])
            pltpu.make_async_copy(tB_h.at[idxB_v.at[i]], bufB[s], sB[s]).wait()
            pltpu.sync_copy(bufB[s], outB.at[pl.ds(off, CH)])

        issue(0, 0)

        def body(g, _):
            for b in range(2):
                i = 2 * g + b
                @pl.when(i + 1 < rows_w)
                def _():
                    issue(i + 1, 1 - b)

                finish(i, b)
            return 0

        lax.fori_loop(0, rows_w // 2, body, 0)

    return k(tA, tB, idxA3, idxB3)


def _sc_scatter_add(m, dst3, n_rows, zslab):
    E, D = m.shape
    rows_w = E // (NW * CH)
    slab = n_rows // NS
    mesh = plsc.VectorSubcoreMesh(core_axis_name="c", subcore_axis_name="s")

    @functools.partial(
        pl.kernel,
        out_type=jax.ShapeDtypeStruct((NC, n_rows, D), jnp.float32),
        mesh=mesh,
        compiler_params=pltpu.CompilerParams(needs_layout_passes=False),
        scratch_types=[
            pltpu.VMEM((rows_w, CH), jnp.int32),
            pltpu.VMEM_SHARED((n_rows, D), jnp.float32),
        ] + [pltpu.VMEM((CH, D), jnp.float32)] * 2
          + [pltpu.SemaphoreType.DMA] * 2,
    )
    def k(m_h, dst_h, z_h, out, idx_v, acc, *bufs):
        nbuf = 2
        buf = bufs[:nbuf]
        sem = bufs[nbuf:2 * nbuf]
        cid = lax.axis_index("c")
        sid = lax.axis_index("s")
        wid = sid * NC + cid
        row0 = wid * rows_w
        pltpu.sync_copy(z_h, acc.at[pl.ds(sid * slab, slab)])
        pltpu.sync_copy(dst_h.at[wid], idx_v)
        plsc.subcore_barrier()

        def l_issue(i, s):
            pltpu.async_copy(m_h.at[pl.ds((row0 + i) * CH, CH)], buf[s], sem[s])

        def l_wait(i, s):
            pltpu.make_async_copy(
                m_h.at[pl.ds((row0 + i) * CH, CH)], buf[s], sem[s]).wait()

        l_issue(0, 0)

        def body(g, _):
            for b in range(nbuf):
                i = nbuf * g + b
                l_wait(i, b)

                @pl.when(i + 1 < rows_w)
                def _():
                    l_issue(i + 1, 1 - b)

                pltpu.sync_copy(buf[b], acc.at[idx_v.at[i]], add=True)
            return 0

        lax.fori_loop(0, rows_w // nbuf, body, 0)
        plsc.subcore_barrier()
        pltpu.sync_copy(acc.at[pl.ds(sid * slab, slab)],
                        out.at[cid, pl.ds(sid * slab, slab)])

    return k(m, dst3, zslab)


def _sc_edge_sumsq(pos4, il3, ir3):
    _, rows_w, _ = il3.shape
    E = NW * rows_w * CH
    per_w = rows_w * CH
    N4 = pos4.shape[0]
    groups = CH // 16
    mesh = plsc.VectorSubcoreMesh(core_axis_name="c", subcore_axis_name="s")

    @functools.partial(
        pl.kernel,
        out_type=jax.ShapeDtypeStruct((E,), jnp.float32),
        mesh=mesh,
        compiler_params=pltpu.CompilerParams(needs_layout_passes=False),
        scratch_types=[
            pltpu.VMEM((N4,), jnp.float32),
            pltpu.VMEM((rows_w, CH), jnp.int32),
            pltpu.VMEM((rows_w, CH), jnp.int32),
            pltpu.VMEM((per_w,), jnp.float32),
        ],
    )
    def k(pos_h, iL_h, iR_h, out, pos_v, iL_v, iR_v, out_v):
        wid = lax.axis_index("s") * NC + lax.axis_index("c")
        pltpu.sync_copy(pos_h, pos_v)
        pltpu.sync_copy(iL_h.at[wid], iL_v)
        pltpu.sync_copy(iR_h.at[wid], iR_v)

        def body(r, _):
            for g in range(groups):
                il = iL_v[r, pl.ds(g * 16, 16)] * 4
                ir = iR_v[r, pl.ds(g * 16, 16)] * 4
                acc = jnp.zeros((16,), jnp.float32)
                for c in range(3):
                    a = plsc.load_gather(pos_v, [il + c])
                    b = plsc.load_gather(pos_v, [ir + c])
                    d = a - b
                    acc = acc + d * d
                out_v[pl.ds(r * CH + g * 16, 16)] = acc
            return 0

        lax.fori_loop(0, rows_w, body, 0)
        pltpu.sync_copy(out_v, out.at[pl.ds(wid * per_w, per_w)])

    return k(pos4, il3, ir3)


def _silu(v):
    return v * jax.nn.sigmoid(v)


def _tc_embed(xg, W_emb, b_emb, WA, WB, nb):
    N, D = xg.shape
    H = W_emb.shape[1]
    grid = N // nb

    def body(x_ref, we, be, wa, wb, h_ref, a_ref, b_ref):
        h = jnp.dot(x_ref[...], we[...],
                    preferred_element_type=jnp.float32) + be[...]
        h_ref[...] = h
        a_ref[...] = jnp.dot(h, wa[...], preferred_element_type=jnp.float32)
        b_ref[...] = jnp.dot(h, wb[...], preferred_element_type=jnp.float32)

    w_spec = [pl.BlockSpec((D, H), lambda i: (0, 0)),
              pl.BlockSpec((1, H), lambda i: (0, 0)),
              pl.BlockSpec((H, H), lambda i: (0, 0)),
              pl.BlockSpec((H, H), lambda i: (0, 0))]
    return pl.pallas_call(
        body,
        grid=(grid,),
        in_specs=[pl.BlockSpec((nb, D), lambda i: (i, 0))] + w_spec,
        out_specs=[pl.BlockSpec((nb, H), lambda i: (i, 0))] * 3,
        out_shape=[jax.ShapeDtypeStruct((N, H), jnp.float32)] * 3,
    )(xg, W_emb, b_emb, WA, WB)


def _tc_message(Asrc, Bdst, sumsq_c, w_inv, bm1, Wm2, bm2, eb):
    E, H = Asrc.shape
    grid = E // eb
    rb = eb // 128

    def body(a_ref, b_ref, s_ref, wi, b1, w2, b2, m_ref):
        invc = jnp.sqrt(s_ref[...] + 1e-12)
        row = lax.broadcasted_iota(jnp.int32, (eb, rb), 0) // 128
        col = lax.broadcasted_iota(jnp.int32, (eb, rb), 1)
        sel = (row == col).astype(jnp.float32)
        t = jnp.dot(sel, invc, preferred_element_type=jnp.float32)
        lane = lax.broadcasted_iota(jnp.int32, (eb, 128), 1)
        rmod = lax.broadcasted_iota(jnp.int32, (eb, 128), 0) % 128
        inv_col = jnp.sum(jnp.where(lane == rmod, t, 0.0), axis=1,
                          keepdims=True)
        z = a_ref[...] + b_ref[...] + inv_col * wi[...] + b1[...]
        z = _silu(z)
        mm = jnp.dot(z, w2[...], preferred_element_type=jnp.float32) + b2[...]
        m_ref[...] = _silu(mm)

    return pl.pallas_call(
        body,
        grid=(grid,),
        in_specs=[
            pl.BlockSpec((eb, H), lambda i: (i, 0)),
            pl.BlockSpec((eb, H), lambda i: (i, 0)),
            pl.BlockSpec((rb, 128), lambda i: (i, 0)),
            pl.BlockSpec((1, H), lambda i: (0, 0)),
            pl.BlockSpec((1, H), lambda i: (0, 0)),
            pl.BlockSpec((H, H), lambda i: (0, 0)),
            pl.BlockSpec((1, H), lambda i: (0, 0)),
        ],
        out_specs=pl.BlockSpec((eb, H), lambda i: (i, 0)),
        out_shape=jax.ShapeDtypeStruct((E, H), jnp.float32),
    )(Asrc, Bdst, sumsq_c, w_inv, bm1, Wm2, bm2)


def _tc_update(h, agg0, agg1, W1h, W1a, b1, W2, b2, WA, WB, nb):
    N, H = h.shape
    grid = N // nb
    n_out = 3 if WA is not None else 1

    def body(h_ref, a0_ref, a1_ref, w1h, w1a, b1r, w2, b2r, *outs):
        h_blk = h_ref[...]
        agg = a0_ref[...] + a1_ref[...]
        u = jnp.dot(h_blk, w1h[...], preferred_element_type=jnp.float32)
        u += jnp.dot(agg, w1a[...], preferred_element_type=jnp.float32)
        u = _silu(u + b1r[...])
        u = jnp.dot(u, w2[...], preferred_element_type=jnp.float32) + b2r[...]
        hn = h_blk + u
        if n_out == 3:
            wa, wb, hn_ref, a_ref, b_ref = outs
            hn_ref[...] = hn
            a_ref[...] = jnp.dot(hn, wa[...],
                                 preferred_element_type=jnp.float32)
            b_ref[...] = jnp.dot(hn, wb[...],
                                 preferred_element_type=jnp.float32)
        else:
            (hn_ref,) = outs
            hn_ref[...] = hn

    in_specs = [
        pl.BlockSpec((nb, H), lambda i: (i, 0)),
        pl.BlockSpec((nb, H), lambda i: (i, 0)),
        pl.BlockSpec((nb, H), lambda i: (i, 0)),
        pl.BlockSpec((H, H), lambda i: (0, 0)),
        pl.BlockSpec((H, H), lambda i: (0, 0)),
        pl.BlockSpec((1, H), lambda i: (0, 0)),
        pl.BlockSpec((H, H), lambda i: (0, 0)),
        pl.BlockSpec((1, H), lambda i: (0, 0)),
    ]
    args = [h, agg0, agg1, W1h, W1a, b1, W2, b2]
    if n_out == 3:
        in_specs += [pl.BlockSpec((H, H), lambda i: (0, 0))] * 2
        args += [WA, WB]
    return pl.pallas_call(
        body,
        grid=(grid,),
        in_specs=in_specs,
        out_specs=[pl.BlockSpec((nb, H), lambda i: (i, 0))] * n_out,
        out_shape=[jax.ShapeDtypeStruct((N, H), jnp.float32)] * n_out,
    )(*args)


def _tc_pool(h, xb_row, Wp1, bp1, Wp2, bp2, Wq1, bq1, Wq2, bq2, n_graphs, nb):
    N, H = h.shape
    grid = N // nb

    def body(h_ref, xb_ref, wp1, bp1r, wp2, bp2r, wq1, bq1r, wq2, bq2r,
             out_ref, acc):
        step = pl.program_id(0)

        @pl.when(step == 0)
        def _():
            acc[...] = jnp.zeros_like(acc)

        hp = _silu(jnp.dot(h_ref[...], wp1[...],
                           preferred_element_type=jnp.float32) + bp1r[...])
        hp = jnp.dot(hp, wp2[...], preferred_element_type=jnp.float32) \
            + bp2r[...]
        rows = lax.broadcasted_iota(jnp.int32, (n_graphs, nb), 0)
        mask = (rows == xb_ref[0]).astype(jnp.float32)
        acc[...] += jnp.dot(mask, hp, preferred_element_type=jnp.float32)

        @pl.when(step == grid - 1)
        def _():
            p = acc[...]
            q = _silu(jnp.dot(p, wq1[...],
                              preferred_element_type=jnp.float32) + bq1r[...])
            out_ref[...] = jnp.dot(q, wq2[...],
                                   preferred_element_type=jnp.float32) \
                + bq2r[...]

    return pl.pallas_call(
        body,
        grid=(grid,),
        in_specs=[
            pl.BlockSpec((nb, H), lambda i: (i, 0)),
            pl.BlockSpec((1, 1, nb), lambda i: (i, 0, 0)),
            pl.BlockSpec((H, H), lambda i: (0, 0)),
            pl.BlockSpec((1, H), lambda i: (0, 0)),
            pl.BlockSpec((H, H), lambda i: (0, 0)),
            pl.BlockSpec((1, H), lambda i: (0, 0)),
            pl.BlockSpec((H, H), lambda i: (0, 0)),
            pl.BlockSpec((1, H), lambda i: (0, 0)),
            pl.BlockSpec((H, 1), lambda i: (0, 0)),
            pl.BlockSpec((1, 1), lambda i: (0, 0)),
        ],
        out_specs=pl.BlockSpec((n_graphs, 1), lambda i: (0, 0)),
        out_shape=jax.ShapeDtypeStruct((n_graphs, 1), jnp.float32),
        scratch_shapes=[pltpu.VMEM((n_graphs, H), jnp.float32)],
    )(h, xb_row, Wp1, bp1, Wp2, bp2, Wq1, bq1, Wq2, bq2)


def kernel(x, pos, feat_ind, adj, inv_ind, x_batch, W_emb, b_emb, Wm1, bm1,
           Wm2, bm2, Wu1, bu1, Wu2, bu2, Wpre1, bpre1, Wpre2, bpre2, Wpost1,
           bpost1, Wpost2, bpost2):
    N, D = x.shape
    H = W_emb.shape[1]
    L = Wm1.shape[0]
    E = adj.shape[1]
    n_graphs = 64

    xg = jnp.take(x, feat_ind, axis=0)
    NP = 10112
    EP = 327680
    pad = EP - E
    rows_w = EP // (NW * CH)
    src3 = jnp.concatenate([adj[0], jnp.zeros((pad,), jnp.int32)]
                           ).reshape(NW, rows_w, CH)
    dst3 = jnp.concatenate([adj[1], jnp.full((pad,), N, jnp.int32)]
                           ).reshape(NW, rows_w, CH)
    il3 = jnp.concatenate([inv_ind[0], jnp.zeros((pad,), jnp.int32)]
                          ).reshape(NW, rows_w, CH)
    ir3 = jnp.concatenate([inv_ind[1], jnp.zeros((pad,), jnp.int32)]
                          ).reshape(NW, rows_w, CH)
    pos4 = jnp.concatenate([pos, jnp.zeros((N, 1), jnp.float32)],
                           axis=1).reshape(-1)
    zslab = jnp.zeros((NP // NS, H), jnp.float32)

    h, A, Bt = _tc_embed(xg, W_emb, b_emb[None], Wm1[0, :H], Wm1[0, H:2 * H],
                         nb=2000)

    sumsq = _sc_edge_sumsq(pos4, il3, ir3).reshape(EP // 128, 128)

    for l in range(L):
        Asrc, Bdst = _sc_gather2(A, Bt, src3, dst3)
        m = _tc_message(Asrc, Bdst, sumsq, Wm1[l, 2 * H:2 * H + 1, :],
                        bm1[l][None], Wm2[l], bm2[l][None], eb=2048)
        aggp = _sc_scatter_add(m, dst3, NP, zslab)
        if l < L - 1:
            h, A, Bt = _tc_update(h, aggp[0], aggp[1], Wu1[l, :H],
                                  Wu1[l, H:], bu1[l][None], Wu2[l],
                                  bu2[l][None], Wm1[l + 1, :H],
                                  Wm1[l + 1, H:2 * H], nb=2000)
        else:
            (h,) = _tc_update(h, aggp[0], aggp[1], Wu1[l, :H], Wu1[l, H:],
                              bu1[l][None], Wu2[l], bu2[l][None],
                              None, None, nb=2000)

    xb3 = x_batch.astype(jnp.int32).reshape(N // 2000, 1, 2000)
    out = _tc_pool(h, xb3, Wpre1, bpre1[None],
                   Wpre2, bpre2[None], Wpost1, bpost1[None], Wpost2,
                   bpost2[None, :], n_graphs, nb=2000)
    return jnp.squeeze(out)


# speedup vs baseline: 1.0986x; 1.0355x over previous
import functools

import jax
import jax.numpy as jnp
from jax import lax
from jax.experimental import pallas as pl
from jax.experimental.pallas import tpu as pltpu
from jax.experimental.pallas import tpu_sc as plsc

NC, NS = 2, 16
NW = NC * NS
CH = 80


def _sc_gather2(tA, tB, idxA3, idxB3):
    _, rows_w, _ = idxA3.shape
    E = NW * rows_w * CH
    DA = tA.shape[1]
    DB = tB.shape[1]
    mesh = plsc.VectorSubcoreMesh(core_axis_name="c", subcore_axis_name="s")

    @functools.partial(
        pl.kernel,
        out_type=[jax.ShapeDtypeStruct((E, DA), jnp.float32),
                  jax.ShapeDtypeStruct((E, DB), jnp.float32)],
        mesh=mesh,
        compiler_params=pltpu.CompilerParams(needs_layout_passes=False),
        scratch_types=[
            pltpu.VMEM((rows_w, CH), jnp.int32),
            pltpu.VMEM((rows_w, CH), jnp.int32),
            pltpu.VMEM((CH, DA), jnp.float32),
            pltpu.VMEM((CH, DA), jnp.float32),
            pltpu.VMEM((CH, DB), jnp.float32),
            pltpu.VMEM((CH, DB), jnp.float32),
            pltpu.SemaphoreType.DMA,
            pltpu.SemaphoreType.DMA,
            pltpu.SemaphoreType.DMA,
            pltpu.SemaphoreType.DMA,
        ],
    )
    def k(tA_h, tB_h, iA_h, iB_h, outA, outB,
          idxA_v, idxB_v, bufA0, bufA1, bufB0, bufB1, sA0, sA1, sB0, sB1):
        wid = lax.axis_index("s") * NC + lax.axis_index("c")
        row0 = wid * rows_w
        pltpu.sync_copy(iA_h.at[wid], idxA_v)
        pltpu.sync_copy(iB_h.at[wid], idxB_v)
        bufA = (bufA0, bufA1)
        bufB = (bufB0, bufB1)
        sA = (sA0, sA1)
        sB = (sB0, sB1)

        def issue(i, s):
            pltpu.async_copy(tA_h.at[idxA_v.at[i]], bufA[s], sA[s])
            pltpu.async_copy(tB_h.at[idxB_v.at[i]], bufB[s], sB[s])

        def finish(i, s):
            off = (row0 + i) * CH
            pltpu.make_async_copy(tA_h.at[idxA_v.at[i]], bufA[s], sA[s]).wait()
            pltpu.sync_copy(bufA[s], outA.at[pl.ds(off, CH)])
            pltpu.make_async_copy(tB_h.at[idxB_v.at[i]], bufB[s], sB[s]).wait()
            pltpu.sync_copy(bufB[s], outB.at[pl.ds(off, CH)])

        issue(0, 0)

        def body(g, _):
            for b in range(2):
                i = 2 * g + b
                @pl.when(i + 1 < rows_w)
                def _():
                    issue(i + 1, 1 - b)

                finish(i, b)
            return 0

        lax.fori_loop(0, rows_w // 2, body, 0)

    return k(tA, tB, idxA3, idxB3)


def _sc_scatter_add(m, dst3, n_rows, zslab):
    E, D = m.shape
    rows_w = E // (NW * CH)
    slab = n_rows // NS
    mesh = plsc.VectorSubcoreMesh(core_axis_name="c", subcore_axis_name="s")

    @functools.partial(
        pl.kernel,
        out_type=jax.ShapeDtypeStruct((NC, n_rows, D), jnp.float32),
        mesh=mesh,
        compiler_params=pltpu.CompilerParams(needs_layout_passes=False),
        scratch_types=[
            pltpu.VMEM((rows_w, CH), jnp.int32),
            pltpu.VMEM_SHARED((n_rows, D), jnp.float32),
        ] + [pltpu.VMEM((CH, D), jnp.float32)] * 2
          + [pltpu.SemaphoreType.DMA] * 2,
    )
    def k(m_h, dst_h, z_h, out, idx_v, acc, *bufs):
        nbuf = 2
        buf = bufs[:nbuf]
        sem = bufs[nbuf:2 * nbuf]
        cid = lax.axis_index("c")
        sid = lax.axis_index("s")
        wid = sid * NC + cid
        row0 = wid * rows_w
        pltpu.sync_copy(z_h, acc.at[pl.ds(sid * slab, slab)])
        pltpu.sync_copy(dst_h.at[wid], idx_v)
        plsc.subcore_barrier()

        def l_issue(i, s):
            pltpu.async_copy(m_h.at[pl.ds((row0 + i) * CH, CH)], buf[s], sem[s])

        def l_wait(i, s):
            pltpu.make_async_copy(
                m_h.at[pl.ds((row0 + i) * CH, CH)], buf[s], sem[s]).wait()

        l_issue(0, 0)

        def body(g, _):
            for b in range(nbuf):
                i = nbuf * g + b

                @pl.when(i + 1 < rows_w)
                def _():
                    l_issue(i + 1, 1 - b)

                l_wait(i, b)
                pltpu.sync_copy(buf[b], acc.at[idx_v.at[i]], add=True)
            return 0

        lax.fori_loop(0, rows_w // nbuf, body, 0)
        plsc.subcore_barrier()
        pltpu.sync_copy(acc.at[pl.ds(sid * slab, slab)],
                        out.at[cid, pl.ds(sid * slab, slab)])

    return k(m, dst3, zslab)


def _sc_edge_sumsq(pos4, il3, ir3):
    _, rows_w, _ = il3.shape
    E = NW * rows_w * CH
    per_w = rows_w * CH
    N4 = pos4.shape[0]
    groups = CH // 16
    mesh = plsc.VectorSubcoreMesh(core_axis_name="c", subcore_axis_name="s")

    @functools.partial(
        pl.kernel,
        out_type=jax.ShapeDtypeStruct((E,), jnp.float32),
        mesh=mesh,
        compiler_params=pltpu.CompilerParams(needs_layout_passes=False),
        scratch_types=[
            pltpu.VMEM((N4,), jnp.float32),
            pltpu.VMEM((rows_w, CH), jnp.int32),
            pltpu.VMEM((rows_w, CH), jnp.int32),
            pltpu.VMEM((per_w,), jnp.float32),
        ],
    )
    def k(pos_h, iL_h, iR_h, out, pos_v, iL_v, iR_v, out_v):
        wid = lax.axis_index("s") * NC + lax.axis_index("c")
        pltpu.sync_copy(pos_h, pos_v)
        pltpu.sync_copy(iL_h.at[wid], iL_v)
        pltpu.sync_copy(iR_h.at[wid], iR_v)

        def body(r, _):
            for g in range(groups):
                il = iL_v[r, pl.ds(g * 16, 16)] * 4
                ir = iR_v[r, pl.ds(g * 16, 16)] * 4
                acc = jnp.zeros((16,), jnp.float32)
                for c in range(3):
                    a = plsc.load_gather(pos_v, [il + c])
                    b = plsc.load_gather(pos_v, [ir + c])
                    d = a - b
                    acc = acc + d * d
                out_v[pl.ds(r * CH + g * 16, 16)] = acc
            return 0

        lax.fori_loop(0, rows_w, body, 0)
        pltpu.sync_copy(out_v, out.at[pl.ds(wid * per_w, per_w)])

    return k(pos4, il3, ir3)


def _silu(v):
    return v * jax.nn.sigmoid(v)


def _tc_embed(xg, W_emb, b_emb, WA, WB, nb):
    N, D = xg.shape
    H = W_emb.shape[1]
    grid = N // nb

    def body(x_ref, we, be, wa, wb, h_ref, a_ref, b_ref):
        h = jnp.dot(x_ref[...], we[...],
                    preferred_element_type=jnp.float32) + be[...]
        h_ref[...] = h
        a_ref[...] = jnp.dot(h, wa[...], preferred_element_type=jnp.float32)
        b_ref[...] = jnp.dot(h, wb[...], preferred_element_type=jnp.float32)

    w_spec = [pl.BlockSpec((D, H), lambda i: (0, 0)),
              pl.BlockSpec((1, H), lambda i: (0, 0)),
              pl.BlockSpec((H, H), lambda i: (0, 0)),
              pl.BlockSpec((H, H), lambda i: (0, 0))]
    return pl.pallas_call(
        body,
        grid=(grid,),
        in_specs=[pl.BlockSpec((nb, D), lambda i: (i, 0))] + w_spec,
        out_specs=[pl.BlockSpec((nb, H), lambda i: (i, 0))] * 3,
        out_shape=[jax.ShapeDtypeStruct((N, H), jnp.float32)] * 3,
    )(xg, W_emb, b_emb, WA, WB)


def _tc_message(Asrc, Bdst, sumsq_c, w_inv, bm1, Wm2, bm2, eb):
    E, H = Asrc.shape
    grid = E // eb
    rb = eb // 128

    def body(a_ref, b_ref, s_ref, wi, b1, w2, b2, m_ref):
        invc = jnp.sqrt(s_ref[...] + 1e-12)
        row = lax.broadcasted_iota(jnp.int32, (eb, rb), 0) // 128
        col = lax.broadcasted_iota(jnp.int32, (eb, rb), 1)
        sel = (row == col).astype(jnp.float32)
        t = jnp.dot(sel, invc, preferred_element_type=jnp.float32)
        lane = lax.broadcasted_iota(jnp.int32, (eb, 128), 1)
        rmod = lax.broadcasted_iota(jnp.int32, (eb, 128), 0) % 128
        inv_col = jnp.sum(jnp.where(lane == rmod, t, 0.0), axis=1,
                          keepdims=True)
        z = a_ref[...] + b_ref[...] + inv_col * wi[...] + b1[...]
        z = _silu(z)
        mm = jnp.dot(z, w2[...], preferred_element_type=jnp.float32) + b2[...]
        m_ref[...] = _silu(mm)

    return pl.pallas_call(
        body,
        grid=(grid,),
        in_specs=[
            pl.BlockSpec((eb, H), lambda i: (i, 0)),
            pl.BlockSpec((eb, H), lambda i: (i, 0)),
            pl.BlockSpec((rb, 128), lambda i: (i, 0)),
            pl.BlockSpec((1, H), lambda i: (0, 0)),
            pl.BlockSpec((1, H), lambda i: (0, 0)),
            pl.BlockSpec((H, H), lambda i: (0, 0)),
            pl.BlockSpec((1, H), lambda i: (0, 0)),
        ],
        out_specs=pl.BlockSpec((eb, H), lambda i: (i, 0)),
        out_shape=jax.ShapeDtypeStruct((E, H), jnp.float32),
    )(Asrc, Bdst, sumsq_c, w_inv, bm1, Wm2, bm2)


def _tc_update(h, agg0, agg1, W1h, W1a, b1, W2, b2, WA, WB, nb):
    N, H = h.shape
    grid = N // nb
    n_out = 3 if WA is not None else 1

    def body(h_ref, a0_ref, a1_ref, w1h, w1a, b1r, w2, b2r, *outs):
        h_blk = h_ref[...]
        agg = a0_ref[...] + a1_ref[...]
        u = jnp.dot(h_blk, w1h[...], preferred_element_type=jnp.float32)
        u += jnp.dot(agg, w1a[...], preferred_element_type=jnp.float32)
        u = _silu(u + b1r[...])
        u = jnp.dot(u, w2[...], preferred_element_type=jnp.float32) + b2r[...]
        hn = h_blk + u
        if n_out == 3:
            wa, wb, hn_ref, a_ref, b_ref = outs
            hn_ref[...] = hn
            a_ref[...] = jnp.dot(hn, wa[...],
                                 preferred_element_type=jnp.float32)
            b_ref[...] = jnp.dot(hn, wb[...],
                                 preferred_element_type=jnp.float32)
        else:
            (hn_ref,) = outs
            hn_ref[...] = hn

    in_specs = [
        pl.BlockSpec((nb, H), lambda i: (i, 0)),
        pl.BlockSpec((nb, H), lambda i: (i, 0)),
        pl.BlockSpec((nb, H), lambda i: (i, 0)),
        pl.BlockSpec((H, H), lambda i: (0, 0)),
        pl.BlockSpec((H, H), lambda i: (0, 0)),
        pl.BlockSpec((1, H), lambda i: (0, 0)),
        pl.BlockSpec((H, H), lambda i: (0, 0)),
        pl.BlockSpec((1, H), lambda i: (0, 0)),
    ]
    args = [h, agg0, agg1, W1h, W1a, b1, W2, b2]
    if n_out == 3:
        in_specs += [pl.BlockSpec((H, H), lambda i: (0, 0))] * 2
        args += [WA, WB]
    return pl.pallas_call(
        body,
        grid=(grid,),
        in_specs=in_specs,
        out_specs=[pl.BlockSpec((nb, H), lambda i: (i, 0))] * n_out,
        out_shape=[jax.ShapeDtypeStruct((N, H), jnp.float32)] * n_out,
    )(*args)


def _tc_pool(h, xb_row, Wp1, bp1, Wp2, bp2, Wq1, bq1, Wq2, bq2, n_graphs, nb):
    N, H = h.shape
    grid = N // nb

    def body(h_ref, xb_ref, wp1, bp1r, wp2, bp2r, wq1, bq1r, wq2, bq2r,
             out_ref, acc):
        step = pl.program_id(0)

        @pl.when(step == 0)
        def _():
            acc[...] = jnp.zeros_like(acc)

        hp = _silu(jnp.dot(h_ref[...], wp1[...],
                           preferred_element_type=jnp.float32) + bp1r[...])
        hp = jnp.dot(hp, wp2[...], preferred_element_type=jnp.float32) \
            + bp2r[...]
        rows = lax.broadcasted_iota(jnp.int32, (n_graphs, nb), 0)
        mask = (rows == xb_ref[0]).astype(jnp.float32)
        acc[...] += jnp.dot(mask, hp, preferred_element_type=jnp.float32)

        @pl.when(step == grid - 1)
        def _():
            p = acc[...]
            q = _silu(jnp.dot(p, wq1[...],
                              preferred_element_type=jnp.float32) + bq1r[...])
            out_ref[...] = jnp.dot(q, wq2[...],
                                   preferred_element_type=jnp.float32) \
                + bq2r[...]

    return pl.pallas_call(
        body,
        grid=(grid,),
        in_specs=[
            pl.BlockSpec((nb, H), lambda i: (i, 0)),
            pl.BlockSpec((1, 1, nb), lambda i: (i, 0, 0)),
            pl.BlockSpec((H, H), lambda i: (0, 0)),
            pl.BlockSpec((1, H), lambda i: (0, 0)),
            pl.BlockSpec((H, H), lambda i: (0, 0)),
            pl.BlockSpec((1, H), lambda i: (0, 0)),
            pl.BlockSpec((H, H), lambda i: (0, 0)),
            pl.BlockSpec((1, H), lambda i: (0, 0)),
            pl.BlockSpec((H, 1), lambda i: (0, 0)),
            pl.BlockSpec((1, 1), lambda i: (0, 0)),
        ],
        out_specs=pl.BlockSpec((n_graphs, 1), lambda i: (0, 0)),
        out_shape=jax.ShapeDtypeStruct((n_graphs, 1), jnp.float32),
        scratch_shapes=[pltpu.VMEM((n_graphs, H), jnp.float32)],
    )(h, xb_row, Wp1, bp1, Wp2, bp2, Wq1, bq1, Wq2, bq2)


def kernel(x, pos, feat_ind, adj, inv_ind, x_batch, W_emb, b_emb, Wm1, bm1,
           Wm2, bm2, Wu1, bu1, Wu2, bu2, Wpre1, bpre1, Wpre2, bpre2, Wpost1,
           bpost1, Wpost2, bpost2):
    N, D = x.shape
    H = W_emb.shape[1]
    L = Wm1.shape[0]
    E = adj.shape[1]
    n_graphs = 64

    xg = jnp.take(x, feat_ind, axis=0)
    NP = 10112
    EP = 327680
    pad = EP - E
    rows_w = EP // (NW * CH)
    src3 = jnp.concatenate([adj[0], jnp.zeros((pad,), jnp.int32)]
                           ).reshape(NW, rows_w, CH)
    dst3 = jnp.concatenate([adj[1], jnp.full((pad,), N, jnp.int32)]
                           ).reshape(NW, rows_w, CH)
    il3 = jnp.concatenate([inv_ind[0], jnp.zeros((pad,), jnp.int32)]
                          ).reshape(NW, rows_w, CH)
    ir3 = jnp.concatenate([inv_ind[1], jnp.zeros((pad,), jnp.int32)]
                          ).reshape(NW, rows_w, CH)
    pos4 = jnp.concatenate([pos, jnp.zeros((N, 1), jnp.float32)],
                           axis=1).reshape(-1)
    zslab = jnp.zeros((NP // NS, H), jnp.float32)

    h, A, Bt = _tc_embed(xg, W_emb, b_emb[None], Wm1[0, :H], Wm1[0, H:2 * H],
                         nb=2000)

    sumsq = _sc_edge_sumsq(pos4, il3, ir3).reshape(EP // 128, 128)

    for l in range(L):
        Asrc, Bdst = _sc_gather2(A, Bt, src3, dst3)
        m = _tc_message(Asrc, Bdst, sumsq, Wm1[l, 2 * H:2 * H + 1, :],
                        bm1[l][None], Wm2[l], bm2[l][None], eb=2048)
        aggp = _sc_scatter_add(m, dst3, NP, zslab)
        if l < L - 1:
            h, A, Bt = _tc_update(h, aggp[0], aggp[1], Wu1[l, :H],
                                  Wu1[l, H:], bu1[l][None], Wu2[l],
                                  bu2[l][None], Wm1[l + 1, :H],
                                  Wm1[l + 1, H:2 * H], nb=2000)
        else:
            (h,) = _tc_update(h, aggp[0], aggp[1], Wu1[l, :H], Wu1[l, H:],
                              bu1[l][None], Wu2[l], bu2[l][None],
                              None, None, nb=2000)

    xb3 = x_batch.astype(jnp.int32).reshape(N // 2000, 1, 2000)
    out = _tc_pool(h, xb3, Wpre1, bpre1[None],
                   Wpre2, bpre2[None], Wpost1, bpost1[None], Wpost2,
                   bpost2[None, :], n_graphs, nb=2000)
    return jnp.squeeze(out)


# speedup vs baseline: 1.1049x; 1.0057x over previous
import functools

import jax
import jax.numpy as jnp
from jax import lax
from jax.experimental import pallas as pl
from jax.experimental.pallas import tpu as pltpu
from jax.experimental.pallas import tpu_sc as plsc

NC, NS = 2, 16
NW = NC * NS
CH = 80


def _sc_gather2(tA, tB, idxA3, idxB3):
    _, rows_w, _ = idxA3.shape
    E = NW * rows_w * CH
    DA = tA.shape[1]
    DB = tB.shape[1]
    mesh = plsc.VectorSubcoreMesh(core_axis_name="c", subcore_axis_name="s")

    @functools.partial(
        pl.kernel,
        out_type=[jax.ShapeDtypeStruct((E, DA), jnp.float32),
                  jax.ShapeDtypeStruct((E, DB), jnp.float32)],
        mesh=mesh,
        compiler_params=pltpu.CompilerParams(needs_layout_passes=False),
        scratch_types=[
            pltpu.VMEM((rows_w, CH), jnp.int32),
            pltpu.VMEM((rows_w, CH), jnp.int32),
            pltpu.VMEM((CH, DA), jnp.float32),
            pltpu.VMEM((CH, DA), jnp.float32),
            pltpu.VMEM((CH, DA), jnp.float32),
            pltpu.VMEM((CH, DB), jnp.float32),
            pltpu.VMEM((CH, DB), jnp.float32),
            pltpu.VMEM((CH, DB), jnp.float32),
            pltpu.SemaphoreType.DMA,
            pltpu.SemaphoreType.DMA,
            pltpu.SemaphoreType.DMA,
            pltpu.SemaphoreType.DMA,
            pltpu.SemaphoreType.DMA,
            pltpu.SemaphoreType.DMA,
        ],
    )
    def k(tA_h, tB_h, iA_h, iB_h, outA, outB,
          idxA_v, idxB_v, bufA0, bufA1, bufA2, bufB0, bufB1, bufB2,
          sA0, sA1, sA2, sB0, sB1, sB2):
        wid = lax.axis_index("s") * NC + lax.axis_index("c")
        row0 = wid * rows_w
        pltpu.sync_copy(iA_h.at[wid], idxA_v)
        pltpu.sync_copy(iB_h.at[wid], idxB_v)
        bufA = (bufA0, bufA1, bufA2)
        bufB = (bufB0, bufB1, bufB2)
        sA = (sA0, sA1, sA2)
        sB = (sB0, sB1, sB2)

        def issue(i, s):
            pltpu.async_copy(tA_h.at[idxA_v.at[i]], bufA[s], sA[s])
            pltpu.async_copy(tB_h.at[idxB_v.at[i]], bufB[s], sB[s])

        def finish(i, s):
            off = (row0 + i) * CH
            pltpu.make_async_copy(tA_h.at[idxA_v.at[i]], bufA[s], sA[s]).wait()
            pltpu.sync_copy(bufA[s], outA.at[pl.ds(off, CH)])
            pltpu.make_async_copy(tB_h.at[idxB_v.at[i]], bufB[s], sB[s]).wait()
            pltpu.sync_copy(bufB[s], outB.at[pl.ds(off, CH)])

        issue(0, 0)
        issue(1, 1)

        def body(g, _):
            for b in range(3):
                i = 3 * g + b
                @pl.when(i + 2 < rows_w)
                def _():
                    issue(i + 2, (b + 2) % 3)

                finish(i, b)
            return 0

        lax.fori_loop(0, rows_w // 3, body, 0)
        for i in range((rows_w // 3) * 3, rows_w):
            finish(i, i % 3)

    return k(tA, tB, idxA3, idxB3)


def _sc_scatter_add(m, dst3, n_rows, zslab):
    E, D = m.shape
    rows_w = E // (NW * CH)
    slab = n_rows // NS
    mesh = plsc.VectorSubcoreMesh(core_axis_name="c", subcore_axis_name="s")

    @functools.partial(
        pl.kernel,
        out_type=jax.ShapeDtypeStruct((NC, n_rows, D), jnp.float32),
        mesh=mesh,
        compiler_params=pltpu.CompilerParams(needs_layout_passes=False),
        scratch_types=[
            pltpu.VMEM((rows_w, CH), jnp.int32),
            pltpu.VMEM_SHARED((n_rows, D), jnp.float32),
        ] + [pltpu.VMEM((CH, D), jnp.float32)] * 2
          + [pltpu.SemaphoreType.DMA] * 2,
    )
    def k(m_h, dst_h, z_h, out, idx_v, acc, *bufs):
        nbuf = 2
        buf = bufs[:nbuf]
        sem = bufs[nbuf:2 * nbuf]
        cid = lax.axis_index("c")
        sid = lax.axis_index("s")
        wid = sid * NC + cid
        row0 = wid * rows_w
        pltpu.sync_copy(z_h, acc.at[pl.ds(sid * slab, slab)])
        pltpu.sync_copy(dst_h.at[wid], idx_v)
        plsc.subcore_barrier()

        def l_issue(i, s):
            pltpu.async_copy(m_h.at[pl.ds((row0 + i) * CH, CH)], buf[s], sem[s])

        def l_wait(i, s):
            pltpu.make_async_copy(
                m_h.at[pl.ds((row0 + i) * CH, CH)], buf[s], sem[s]).wait()

        l_issue(0, 0)

        def body(g, _):
            for b in range(nbuf):
                i = nbuf * g + b

                @pl.when(i + 1 < rows_w)
                def _():
                    l_issue(i + 1, 1 - b)

                l_wait(i, b)
                pltpu.sync_copy(buf[b], acc.at[idx_v.at[i]], add=True)
            return 0

        lax.fori_loop(0, rows_w // nbuf, body, 0)
        plsc.subcore_barrier()
        pltpu.sync_copy(acc.at[pl.ds(sid * slab, slab)],
                        out.at[cid, pl.ds(sid * slab, slab)])

    return k(m, dst3, zslab)


def _sc_edge_sumsq(pos4, il3, ir3):
    _, rows_w, _ = il3.shape
    E = NW * rows_w * CH
    per_w = rows_w * CH
    N4 = pos4.shape[0]
    groups = CH // 16
    mesh = plsc.VectorSubcoreMesh(core_axis_name="c", subcore_axis_name="s")

    @functools.partial(
        pl.kernel,
        out_type=jax.ShapeDtypeStruct((E,), jnp.float32),
        mesh=mesh,
        compiler_params=pltpu.CompilerParams(needs_layout_passes=False),
        scratch_types=[
            pltpu.VMEM((N4,), jnp.float32),
            pltpu.VMEM((rows_w, CH), jnp.int32),
            pltpu.VMEM((rows_w, CH), jnp.int32),
            pltpu.VMEM((per_w,), jnp.float32),
        ],
    )
    def k(pos_h, iL_h, iR_h, out, pos_v, iL_v, iR_v, out_v):
        wid = lax.axis_index("s") * NC + lax.axis_index("c")
        pltpu.sync_copy(pos_h, pos_v)
        pltpu.sync_copy(iL_h.at[wid], iL_v)
        pltpu.sync_copy(iR_h.at[wid], iR_v)

        def body(r, _):
            for g in range(groups):
                il = iL_v[r, pl.ds(g * 16, 16)] * 4
                ir = iR_v[r, pl.ds(g * 16, 16)] * 4
                acc = jnp.zeros((16,), jnp.float32)
                for c in range(3):
                    a = plsc.load_gather(pos_v, [il + c])
                    b = plsc.load_gather(pos_v, [ir + c])
                    d = a - b
                    acc = acc + d * d
                out_v[pl.ds(r * CH + g * 16, 16)] = acc
            return 0

        lax.fori_loop(0, rows_w, body, 0)
        pltpu.sync_copy(out_v, out.at[pl.ds(wid * per_w, per_w)])

    return k(pos4, il3, ir3)


def _silu(v):
    return v * jax.nn.sigmoid(v)


def _tc_embed(xg, W_emb, b_emb, WA, WB, nb):
    N, D = xg.shape
    H = W_emb.shape[1]
    grid = N // nb

    def body(x_ref, we, be, wa, wb, h_ref, a_ref, b_ref):
        h = jnp.dot(x_ref[...], we[...],
                    preferred_element_type=jnp.float32) + be[...]
        h_ref[...] = h
        a_ref[...] = jnp.dot(h, wa[...], preferred_element_type=jnp.float32)
        b_ref[...] = jnp.dot(h, wb[...], preferred_element_type=jnp.float32)

    w_spec = [pl.BlockSpec((D, H), lambda i: (0, 0)),
              pl.BlockSpec((1, H), lambda i: (0, 0)),
              pl.BlockSpec((H, H), lambda i: (0, 0)),
              pl.BlockSpec((H, H), lambda i: (0, 0))]
    return pl.pallas_call(
        body,
        grid=(grid,),
        in_specs=[pl.BlockSpec((nb, D), lambda i: (i, 0))] + w_spec,
        out_specs=[pl.BlockSpec((nb, H), lambda i: (i, 0))] * 3,
        out_shape=[jax.ShapeDtypeStruct((N, H), jnp.float32)] * 3,
    )(xg, W_emb, b_emb, WA, WB)


def _tc_message(Asrc, Bdst, sumsq_c, w_inv, bm1, Wm2, bm2, eb):
    E, H = Asrc.shape
    grid = E // eb
    rb = eb // 128

    def body(a_ref, b_ref, s_ref, wi, b1, w2, b2, m_ref):
        invc = jnp.sqrt(s_ref[...] + 1e-12)
        row = lax.broadcasted_iota(jnp.int32, (eb, rb), 0) // 128
        col = lax.broadcasted_iota(jnp.int32, (eb, rb), 1)
        sel = (row == col).astype(jnp.float32)
        t = jnp.dot(sel, invc, preferred_element_type=jnp.float32)
        lane = lax.broadcasted_iota(jnp.int32, (eb, 128), 1)
        rmod = lax.broadcasted_iota(jnp.int32, (eb, 128), 0) % 128
        inv_col = jnp.sum(jnp.where(lane == rmod, t, 0.0), axis=1,
                          keepdims=True)
        z = a_ref[...] + b_ref[...] + inv_col * wi[...] + b1[...]
        z = _silu(z)
        mm = jnp.dot(z, w2[...], preferred_element_type=jnp.float32) + b2[...]
        m_ref[...] = _silu(mm)

    return pl.pallas_call(
        body,
        grid=(grid,),
        in_specs=[
            pl.BlockSpec((eb, H), lambda i: (i, 0)),
            pl.BlockSpec((eb, H), lambda i: (i, 0)),
            pl.BlockSpec((rb, 128), lambda i: (i, 0)),
            pl.BlockSpec((1, H), lambda i: (0, 0)),
            pl.BlockSpec((1, H), lambda i: (0, 0)),
            pl.BlockSpec((H, H), lambda i: (0, 0)),
            pl.BlockSpec((1, H), lambda i: (0, 0)),
        ],
        out_specs=pl.BlockSpec((eb, H), lambda i: (i, 0)),
        out_shape=jax.ShapeDtypeStruct((E, H), jnp.float32),
    )(Asrc, Bdst, sumsq_c, w_inv, bm1, Wm2, bm2)


def _tc_update(h, agg0, agg1, W1h, W1a, b1, W2, b2, WA, WB, nb):
    N, H = h.shape
    grid = N // nb
    n_out = 3 if WA is not None else 1

    def body(h_ref, a0_ref, a1_ref, w1h, w1a, b1r, w2, b2r, *outs):
        h_blk = h_ref[...]
        agg = a0_ref[...] + a1_ref[...]
        u = jnp.dot(h_blk, w1h[...], preferred_element_type=jnp.float32)
        u += jnp.dot(agg, w1a[...], preferred_element_type=jnp.float32)
        u = _silu(u + b1r[...])
        u = jnp.dot(u, w2[...], preferred_element_type=jnp.float32) + b2r[...]
        hn = h_blk + u
        if n_out == 3:
            wa, wb, hn_ref, a_ref, b_ref = outs
            hn_ref[...] = hn
            a_ref[...] = jnp.dot(hn, wa[...],
                                 preferred_element_type=jnp.float32)
            b_ref[...] = jnp.dot(hn, wb[...],
                                 preferred_element_type=jnp.float32)
        else:
            (hn_ref,) = outs
            hn_ref[...] = hn

    in_specs = [
        pl.BlockSpec((nb, H), lambda i: (i, 0)),
        pl.BlockSpec((nb, H), lambda i: (i, 0)),
        pl.BlockSpec((nb, H), lambda i: (i, 0)),
        pl.BlockSpec((H, H), lambda i: (0, 0)),
        pl.BlockSpec((H, H), lambda i: (0, 0)),
        pl.BlockSpec((1, H), lambda i: (0, 0)),
        pl.BlockSpec((H, H), lambda i: (0, 0)),
        pl.BlockSpec((1, H), lambda i: (0, 0)),
    ]
    args = [h, agg0, agg1, W1h, W1a, b1, W2, b2]
    if n_out == 3:
        in_specs += [pl.BlockSpec((H, H), lambda i: (0, 0))] * 2
        args += [WA, WB]
    return pl.pallas_call(
        body,
        grid=(grid,),
        in_specs=in_specs,
        out_specs=[pl.BlockSpec((nb, H), lambda i: (i, 0))] * n_out,
        out_shape=[jax.ShapeDtypeStruct((N, H), jnp.float32)] * n_out,
    )(*args)


def _tc_pool(h, xb_row, Wp1, bp1, Wp2, bp2, Wq1, bq1, Wq2, bq2, n_graphs, nb):
    N, H = h.shape
    grid = N // nb

    def body(h_ref, xb_ref, wp1, bp1r, wp2, bp2r, wq1, bq1r, wq2, bq2r,
             out_ref, acc):
        step = pl.program_id(0)

        @pl.when(step == 0)
        def _():
            acc[...] = jnp.zeros_like(acc)

        hp = _silu(jnp.dot(h_ref[...], wp1[...],
                           preferred_element_type=jnp.float32) + bp1r[...])
        hp = jnp.dot(hp, wp2[...], preferred_element_type=jnp.float32) \
            + bp2r[...]
        rows = lax.broadcasted_iota(jnp.int32, (n_graphs, nb), 0)
        mask = (rows == xb_ref[0]).astype(jnp.float32)
        acc[...] += jnp.dot(mask, hp, preferred_element_type=jnp.float32)

        @pl.when(step == grid - 1)
        def _():
            p = acc[...]
            q = _silu(jnp.dot(p, wq1[...],
                              preferred_element_type=jnp.float32) + bq1r[...])
            out_ref[...] = jnp.dot(q, wq2[...],
                                   preferred_element_type=jnp.float32) \
                + bq2r[...]

    return pl.pallas_call(
        body,
        grid=(grid,),
        in_specs=[
            pl.BlockSpec((nb, H), lambda i: (i, 0)),
            pl.BlockSpec((1, 1, nb), lambda i: (i, 0, 0)),
            pl.BlockSpec((H, H), lambda i: (0, 0)),
            pl.BlockSpec((1, H), lambda i: (0, 0)),
            pl.BlockSpec((H, H), lambda i: (0, 0)),
            pl.BlockSpec((1, H), lambda i: (0, 0)),
            pl.BlockSpec((H, H), lambda i: (0, 0)),
            pl.BlockSpec((1, H), lambda i: (0, 0)),
            pl.BlockSpec((H, 1), lambda i: (0, 0)),
            pl.BlockSpec((1, 1), lambda i: (0, 0)),
        ],
        out_specs=pl.BlockSpec((n_graphs, 1), lambda i: (0, 0)),
        out_shape=jax.ShapeDtypeStruct((n_graphs, 1), jnp.float32),
        scratch_shapes=[pltpu.VMEM((n_graphs, H), jnp.float32)],
    )(h, xb_row, Wp1, bp1, Wp2, bp2, Wq1, bq1, Wq2, bq2)


def kernel(x, pos, feat_ind, adj, inv_ind, x_batch, W_emb, b_emb, Wm1, bm1,
           Wm2, bm2, Wu1, bu1, Wu2, bu2, Wpre1, bpre1, Wpre2, bpre2, Wpost1,
           bpost1, Wpost2, bpost2):
    N, D = x.shape
    H = W_emb.shape[1]
    L = Wm1.shape[0]
    E = adj.shape[1]
    n_graphs = 64

    xg = jnp.take(x, feat_ind, axis=0)
    NP = 10112
    EP = 327680
    pad = EP - E
    rows_w = EP // (NW * CH)
    src3 = jnp.concatenate([adj[0], jnp.zeros((pad,), jnp.int32)]
                           ).reshape(NW, rows_w, CH)
    dst3 = jnp.concatenate([adj[1], jnp.full((pad,), N, jnp.int32)]
                           ).reshape(NW, rows_w, CH)
    il3 = jnp.concatenate([inv_ind[0], jnp.zeros((pad,), jnp.int32)]
                          ).reshape(NW, rows_w, CH)
    ir3 = jnp.concatenate([inv_ind[1], jnp.zeros((pad,), jnp.int32)]
                          ).reshape(NW, rows_w, CH)
    pos4 = jnp.concatenate([pos, jnp.zeros((N, 1), jnp.float32)],
                           axis=1).reshape(-1)
    zslab = jnp.zeros((NP // NS, H), jnp.float32)

    h, A, Bt = _tc_embed(xg, W_emb, b_emb[None], Wm1[0, :H], Wm1[0, H:2 * H],
                         nb=2000)

    sumsq = _sc_edge_sumsq(pos4, il3, ir3).reshape(EP // 128, 128)

    for l in range(L):
        Asrc, Bdst = _sc_gather2(A, Bt, src3, dst3)
        m = _tc_message(Asrc, Bdst, sumsq, Wm1[l, 2 * H:2 * H + 1, :],
                        bm1[l][None], Wm2[l], bm2[l][None], eb=2048)
        aggp = _sc_scatter_add(m, dst3, NP, zslab)
        if l < L - 1:
            h, A, Bt = _tc_update(h, aggp[0], aggp[1], Wu1[l, :H],
                                  Wu1[l, H:], bu1[l][None], Wu2[l],
                                  bu2[l][None], Wm1[l + 1, :H],
                                  Wm1[l + 1, H:2 * H], nb=2000)
        else:
            (h,) = _tc_update(h, aggp[0], aggp[1], Wu1[l, :H], Wu1[l, H:],
                              bu1[l][None], Wu2[l], bu2[l][None],
                              None, None, nb=2000)

    xb3 = x_batch.astype(jnp.int32).reshape(N // 2000, 1, 2000)
    out = _tc_pool(h, xb3, Wpre1, bpre1[None],
                   Wpre2, bpre2[None], Wpost1, bpost1[None], Wpost2,
                   bpost2[None, :], n_graphs, nb=2000)
    return jnp.squeeze(out)


# speedup vs baseline: 1.1290x; 1.0218x over previous
import functools

import jax
import jax.numpy as jnp
from jax import lax
from jax.experimental import pallas as pl
from jax.experimental.pallas import tpu as pltpu
from jax.experimental.pallas import tpu_sc as plsc

NC, NS = 2, 16
NW = NC * NS
CH = 80


def _sc_gather2(tA, tB, idxA3, idxB3):
    _, rows_w, _ = idxA3.shape
    E = NW * rows_w * CH
    DA = tA.shape[1]
    DB = tB.shape[1]
    mesh = plsc.VectorSubcoreMesh(core_axis_name="c", subcore_axis_name="s")

    @functools.partial(
        pl.kernel,
        out_type=[jax.ShapeDtypeStruct((E, DA), jnp.float32),
                  jax.ShapeDtypeStruct((E, DB), jnp.float32)],
        mesh=mesh,
        compiler_params=pltpu.CompilerParams(needs_layout_passes=False),
        scratch_types=[
            pltpu.VMEM((rows_w, CH), jnp.int32),
            pltpu.VMEM((rows_w, CH), jnp.int32),
            pltpu.VMEM((CH, DA), jnp.float32),
            pltpu.VMEM((CH, DA), jnp.float32),
            pltpu.VMEM((CH, DA), jnp.float32),
            pltpu.VMEM((CH, DB), jnp.float32),
            pltpu.VMEM((CH, DB), jnp.float32),
            pltpu.VMEM((CH, DB), jnp.float32),
            pltpu.SemaphoreType.DMA,
            pltpu.SemaphoreType.DMA,
            pltpu.SemaphoreType.DMA,
            pltpu.SemaphoreType.DMA,
            pltpu.SemaphoreType.DMA,
            pltpu.SemaphoreType.DMA,
        ],
    )
    def k(tA_h, tB_h, iA_h, iB_h, outA, outB,
          idxA_v, idxB_v, bufA0, bufA1, bufA2, bufB0, bufB1, bufB2,
          sA0, sA1, sA2, sB0, sB1, sB2):
        wid = lax.axis_index("s") * NC + lax.axis_index("c")
        row0 = wid * rows_w
        pltpu.sync_copy(iA_h.at[wid], idxA_v)
        pltpu.sync_copy(iB_h.at[wid], idxB_v)
        bufA = (bufA0, bufA1, bufA2)
        bufB = (bufB0, bufB1, bufB2)
        sA = (sA0, sA1, sA2)
        sB = (sB0, sB1, sB2)

        def issue(i, s):
            pltpu.async_copy(tA_h.at[idxA_v.at[i]], bufA[s], sA[s])
            pltpu.async_copy(tB_h.at[idxB_v.at[i]], bufB[s], sB[s])

        def finish(i, s):
            off = (row0 + i) * CH
            pltpu.make_async_copy(tA_h.at[idxA_v.at[i]], bufA[s], sA[s]).wait()
            pltpu.sync_copy(bufA[s], outA.at[pl.ds(off, CH)])
            pltpu.make_async_copy(tB_h.at[idxB_v.at[i]], bufB[s], sB[s]).wait()
            pltpu.sync_copy(bufB[s], outB.at[pl.ds(off, CH)])

        issue(0, 0)
        issue(1, 1)

        def body(g, _):
            for b in range(3):
                i = 3 * g + b
                @pl.when(i + 2 < rows_w)
                def _():
                    issue(i + 2, (b + 2) % 3)

                finish(i, b)
            return 0

        lax.fori_loop(0, rows_w // 3, body, 0)
        for i in range((rows_w // 3) * 3, rows_w):
            finish(i, i % 3)

    return k(tA, tB, idxA3, idxB3)


def _sc_scatter_add(m, dst3, n_rows, zslab):
    E, D = m.shape
    rows_w = E // (NW * CH)
    slab = n_rows // NS
    mesh = plsc.VectorSubcoreMesh(core_axis_name="c", subcore_axis_name="s")

    @functools.partial(
        pl.kernel,
        out_type=jax.ShapeDtypeStruct((NC, n_rows, D), jnp.float32),
        mesh=mesh,
        compiler_params=pltpu.CompilerParams(needs_layout_passes=False),
        scratch_types=[
            pltpu.VMEM((rows_w, CH), jnp.int32),
            pltpu.VMEM_SHARED((n_rows, D), jnp.float32),
        ] + [pltpu.VMEM((CH, D), jnp.float32)] * 3
          + [pltpu.SemaphoreType.DMA] * 3,
    )
    def k(m_h, dst_h, z_h, out, idx_v, acc, *bufs):
        nbuf = 3
        buf = bufs[:nbuf]
        sem = bufs[nbuf:2 * nbuf]
        cid = lax.axis_index("c")
        sid = lax.axis_index("s")
        wid = sid * NC + cid
        row0 = wid * rows_w
        pltpu.sync_copy(z_h, acc.at[pl.ds(sid * slab, slab)])
        pltpu.sync_copy(dst_h.at[wid], idx_v)
        plsc.subcore_barrier()

        def l_issue(i, s):
            pltpu.async_copy(m_h.at[pl.ds((row0 + i) * CH, CH)], buf[s], sem[s])

        def l_wait(i, s):
            pltpu.make_async_copy(
                m_h.at[pl.ds((row0 + i) * CH, CH)], buf[s], sem[s]).wait()

        l_issue(0, 0)
        l_issue(1, 1)

        def body(g, _):
            for b in range(nbuf):
                i = nbuf * g + b

                @pl.when(i + 2 < rows_w)
                def _():
                    l_issue(i + 2, (b + 2) % nbuf)

                l_wait(i, b)
                pltpu.sync_copy(buf[b], acc.at[idx_v.at[i]], add=True)
            return 0

        lax.fori_loop(0, rows_w // nbuf, body, 0)
        for i in range((rows_w // nbuf) * nbuf, rows_w):
            l_wait(i, i % nbuf)
            pltpu.sync_copy(buf[i % nbuf], acc.at[idx_v.at[i]], add=True)
        plsc.subcore_barrier()
        pltpu.sync_copy(acc.at[pl.ds(sid * slab, slab)],
                        out.at[cid, pl.ds(sid * slab, slab)])

    return k(m, dst3, zslab)


def _sc_edge_sumsq(pos4, il3, ir3):
    _, rows_w, _ = il3.shape
    E = NW * rows_w * CH
    per_w = rows_w * CH
    N4 = pos4.shape[0]
    groups = CH // 16
    mesh = plsc.VectorSubcoreMesh(core_axis_name="c", subcore_axis_name="s")

    @functools.partial(
        pl.kernel,
        out_type=jax.ShapeDtypeStruct((E,), jnp.float32),
        mesh=mesh,
        compiler_params=pltpu.CompilerParams(needs_layout_passes=False),
        scratch_types=[
            pltpu.VMEM((N4,), jnp.float32),
            pltpu.VMEM((rows_w, CH), jnp.int32),
            pltpu.VMEM((rows_w, CH), jnp.int32),
            pltpu.VMEM((per_w,), jnp.float32),
        ],
    )
    def k(pos_h, iL_h, iR_h, out, pos_v, iL_v, iR_v, out_v):
        wid = lax.axis_index("s") * NC + lax.axis_index("c")
        pltpu.sync_copy(pos_h, pos_v)
        pltpu.sync_copy(iL_h.at[wid], iL_v)
        pltpu.sync_copy(iR_h.at[wid], iR_v)

        def body(r, _):
            for g in range(groups):
                il = iL_v[r, pl.ds(g * 16, 16)] * 4
                ir = iR_v[r, pl.ds(g * 16, 16)] * 4
                acc = jnp.zeros((16,), jnp.float32)
                for c in range(3):
                    a = plsc.load_gather(pos_v, [il + c])
                    b = plsc.load_gather(pos_v, [ir + c])
                    d = a - b
                    acc = acc + d * d
                out_v[pl.ds(r * CH + g * 16, 16)] = acc
            return 0

        lax.fori_loop(0, rows_w, body, 0)
        pltpu.sync_copy(out_v, out.at[pl.ds(wid * per_w, per_w)])

    return k(pos4, il3, ir3)


def _silu(v):
    return v * jax.nn.sigmoid(v)


def _tc_embed(xg, W_emb, b_emb, WA, WB, nb):
    N, D = xg.shape
    H = W_emb.shape[1]
    grid = N // nb

    def body(x_ref, we, be, wa, wb, h_ref, a_ref, b_ref):
        h = jnp.dot(x_ref[...], we[...],
                    preferred_element_type=jnp.float32) + be[...]
        h_ref[...] = h
        a_ref[...] = jnp.dot(h, wa[...], preferred_element_type=jnp.float32)
        b_ref[...] = jnp.dot(h, wb[...], preferred_element_type=jnp.float32)

    w_spec = [pl.BlockSpec((D, H), lambda i: (0, 0)),
              pl.BlockSpec((1, H), lambda i: (0, 0)),
              pl.BlockSpec((H, H), lambda i: (0, 0)),
              pl.BlockSpec((H, H), lambda i: (0, 0))]
    return pl.pallas_call(
        body,
        grid=(grid,),
        in_specs=[pl.BlockSpec((nb, D), lambda i: (i, 0))] + w_spec,
        out_specs=[pl.BlockSpec((nb, H), lambda i: (i, 0))] * 3,
        out_shape=[jax.ShapeDtypeStruct((N, H), jnp.float32)] * 3,
    )(xg, W_emb, b_emb, WA, WB)


def _tc_message(Asrc, Bdst, sumsq_c, w_inv, bm1, Wm2, bm2, eb):
    E, H = Asrc.shape
    grid = E // eb
    rb = eb // 128

    def body(a_ref, b_ref, s_ref, wi, b1, w2, b2, m_ref):
        invc = jnp.sqrt(s_ref[...] + 1e-12)
        row = lax.broadcasted_iota(jnp.int32, (eb, rb), 0) // 128
        col = lax.broadcasted_iota(jnp.int32, (eb, rb), 1)
        sel = (row == col).astype(jnp.float32)
        t = jnp.dot(sel, invc, preferred_element_type=jnp.float32)
        lane = lax.broadcasted_iota(jnp.int32, (eb, 128), 1)
        rmod = lax.broadcasted_iota(jnp.int32, (eb, 128), 0) % 128
        inv_col = jnp.sum(jnp.where(lane == rmod, t, 0.0), axis=1,
                          keepdims=True)
        z = a_ref[...] + b_ref[...] + inv_col * wi[...] + b1[...]
        z = _silu(z)
        mm = jnp.dot(z, w2[...], preferred_element_type=jnp.float32) + b2[...]
        m_ref[...] = _silu(mm)

    return pl.pallas_call(
        body,
        grid=(grid,),
        in_specs=[
            pl.BlockSpec((eb, H), lambda i: (i, 0)),
            pl.BlockSpec((eb, H), lambda i: (i, 0)),
            pl.BlockSpec((rb, 128), lambda i: (i, 0)),
            pl.BlockSpec((1, H), lambda i: (0, 0)),
            pl.BlockSpec((1, H), lambda i: (0, 0)),
            pl.BlockSpec((H, H), lambda i: (0, 0)),
            pl.BlockSpec((1, H), lambda i: (0, 0)),
        ],
        out_specs=pl.BlockSpec((eb, H), lambda i: (i, 0)),
        out_shape=jax.ShapeDtypeStruct((E, H), jnp.float32),
    )(Asrc, Bdst, sumsq_c, w_inv, bm1, Wm2, bm2)


def _tc_update(h, agg0, agg1, W1h, W1a, b1, W2, b2, WA, WB, nb):
    N, H = h.shape
    grid = N // nb
    n_out = 3 if WA is not None else 1

    def body(h_ref, a0_ref, a1_ref, w1h, w1a, b1r, w2, b2r, *outs):
        h_blk = h_ref[...]
        agg = a0_ref[...] + a1_ref[...]
        u = jnp.dot(h_blk, w1h[...], preferred_element_type=jnp.float32)
        u += jnp.dot(agg, w1a[...], preferred_element_type=jnp.float32)
        u = _silu(u + b1r[...])
        u = jnp.dot(u, w2[...], preferred_element_type=jnp.float32) + b2r[...]
        hn = h_blk + u
        if n_out == 3:
            wa, wb, hn_ref, a_ref, b_ref = outs
            hn_ref[...] = hn
            a_ref[...] = jnp.dot(hn, wa[...],
                                 preferred_element_type=jnp.float32)
            b_ref[...] = jnp.dot(hn, wb[...],
                                 preferred_element_type=jnp.float32)
        else:
            (hn_ref,) = outs
            hn_ref[...] = hn

    in_specs = [
        pl.BlockSpec((nb, H), lambda i: (i, 0)),
        pl.BlockSpec((nb, H), lambda i: (i, 0)),
        pl.BlockSpec((nb, H), lambda i: (i, 0)),
        pl.BlockSpec((H, H), lambda i: (0, 0)),
        pl.BlockSpec((H, H), lambda i: (0, 0)),
        pl.BlockSpec((1, H), lambda i: (0, 0)),
        pl.BlockSpec((H, H), lambda i: (0, 0)),
        pl.BlockSpec((1, H), lambda i: (0, 0)),
    ]
    args = [h, agg0, agg1, W1h, W1a, b1, W2, b2]
    if n_out == 3:
        in_specs += [pl.BlockSpec((H, H), lambda i: (0, 0))] * 2
        args += [WA, WB]
    return pl.pallas_call(
        body,
        grid=(grid,),
        in_specs=in_specs,
        out_specs=[pl.BlockSpec((nb, H), lambda i: (i, 0))] * n_out,
        out_shape=[jax.ShapeDtypeStruct((N, H), jnp.float32)] * n_out,
    )(*args)


def _tc_pool(h, xb_row, Wp1, bp1, Wp2, bp2, Wq1, bq1, Wq2, bq2, n_graphs, nb):
    N, H = h.shape
    grid = N // nb

    def body(h_ref, xb_ref, wp1, bp1r, wp2, bp2r, wq1, bq1r, wq2, bq2r,
             out_ref, acc):
        step = pl.program_id(0)

        @pl.when(step == 0)
        def _():
            acc[...] = jnp.zeros_like(acc)

        hp = _silu(jnp.dot(h_ref[...], wp1[...],
                           preferred_element_type=jnp.float32) + bp1r[...])
        hp = jnp.dot(hp, wp2[...], preferred_element_type=jnp.float32) \
            + bp2r[...]
        rows = lax.broadcasted_iota(jnp.int32, (n_graphs, nb), 0)
        mask = (rows == xb_ref[0]).astype(jnp.float32)
        acc[...] += jnp.dot(mask, hp, preferred_element_type=jnp.float32)

        @pl.when(step == grid - 1)
        def _():
            p = acc[...]
            q = _silu(jnp.dot(p, wq1[...],
                              preferred_element_type=jnp.float32) + bq1r[...])
            out_ref[...] = jnp.dot(q, wq2[...],
                                   preferred_element_type=jnp.float32) \
                + bq2r[...]

    return pl.pallas_call(
        body,
        grid=(grid,),
        in_specs=[
            pl.BlockSpec((nb, H), lambda i: (i, 0)),
            pl.BlockSpec((1, 1, nb), lambda i: (i, 0, 0)),
            pl.BlockSpec((H, H), lambda i: (0, 0)),
            pl.BlockSpec((1, H), lambda i: (0, 0)),
            pl.BlockSpec((H, H), lambda i: (0, 0)),
            pl.BlockSpec((1, H), lambda i: (0, 0)),
            pl.BlockSpec((H, H), lambda i: (0, 0)),
            pl.BlockSpec((1, H), lambda i: (0, 0)),
            pl.BlockSpec((H, 1), lambda i: (0, 0)),
            pl.BlockSpec((1, 1), lambda i: (0, 0)),
        ],
        out_specs=pl.BlockSpec((n_graphs, 1), lambda i: (0, 0)),
        out_shape=jax.ShapeDtypeStruct((n_graphs, 1), jnp.float32),
        scratch_shapes=[pltpu.VMEM((n_graphs, H), jnp.float32)],
    )(h, xb_row, Wp1, bp1, Wp2, bp2, Wq1, bq1, Wq2, bq2)


def kernel(x, pos, feat_ind, adj, inv_ind, x_batch, W_emb, b_emb, Wm1, bm1,
           Wm2, bm2, Wu1, bu1, Wu2, bu2, Wpre1, bpre1, Wpre2, bpre2, Wpost1,
           bpost1, Wpost2, bpost2):
    N, D = x.shape
    H = W_emb.shape[1]
    L = Wm1.shape[0]
    E = adj.shape[1]
    n_graphs = 64

    xg = jnp.take(x, feat_ind, axis=0)
    NP = 10112
    EP = 327680
    pad = EP - E
    rows_w = EP // (NW * CH)
    src3 = jnp.concatenate([adj[0], jnp.zeros((pad,), jnp.int32)]
                           ).reshape(NW, rows_w, CH)
    dst3 = jnp.concatenate([adj[1], jnp.full((pad,), N, jnp.int32)]
                           ).reshape(NW, rows_w, CH)
    il3 = jnp.concatenate([inv_ind[0], jnp.zeros((pad,), jnp.int32)]
                          ).reshape(NW, rows_w, CH)
    ir3 = jnp.concatenate([inv_ind[1], jnp.zeros((pad,), jnp.int32)]
                          ).reshape(NW, rows_w, CH)
    pos4 = jnp.concatenate([pos, jnp.zeros((N, 1), jnp.float32)],
                           axis=1).reshape(-1)
    zslab = jnp.zeros((NP // NS, H), jnp.float32)

    h, A, Bt = _tc_embed(xg, W_emb, b_emb[None], Wm1[0, :H], Wm1[0, H:2 * H],
                         nb=2000)

    sumsq = _sc_edge_sumsq(pos4, il3, ir3).reshape(EP // 128, 128)

    for l in range(L):
        Asrc, Bdst = _sc_gather2(A, Bt, src3, dst3)
        m = _tc_message(Asrc, Bdst, sumsq, Wm1[l, 2 * H:2 * H + 1, :],
                        bm1[l][None], Wm2[l], bm2[l][None], eb=2048)
        aggp = _sc_scatter_add(m, dst3, NP, zslab)
        if l < L - 1:
            h, A, Bt = _tc_update(h, aggp[0], aggp[1], Wu1[l, :H],
                                  Wu1[l, H:], bu1[l][None], Wu2[l],
                                  bu2[l][None], Wm1[l + 1, :H],
                                  Wm1[l + 1, H:2 * H], nb=2000)
        else:
            (h,) = _tc_update(h, aggp[0], aggp[1], Wu1[l, :H], Wu1[l, H:],
                              bu1[l][None], Wu2[l], bu2[l][None],
                              None, None, nb=2000)

    xb3 = x_batch.astype(jnp.int32).reshape(N // 2000, 1, 2000)
    out = _tc_pool(h, xb3, Wpre1, bpre1[None],
                   Wpre2, bpre2[None], Wpost1, bpost1[None], Wpost2,
                   bpost2[None, :], n_graphs, nb=2000)
    return jnp.squeeze(out)


# speedup vs baseline: 1.1298x; 1.0008x over previous
import functools

import jax
import jax.numpy as jnp
from jax import lax
from jax.experimental import pallas as pl
from jax.experimental.pallas import tpu as pltpu
from jax.experimental.pallas import tpu_sc as plsc

NC, NS = 2, 16
NW = NC * NS
CH = 80


def _sc_gather2(tA, tB, idxA3, idxB3):
    _, rows_w, _ = idxA3.shape
    E = NW * rows_w * CH
    DA = tA.shape[1]
    DB = tB.shape[1]
    mesh = plsc.VectorSubcoreMesh(core_axis_name="c", subcore_axis_name="s")

    @functools.partial(
        pl.kernel,
        out_type=[jax.ShapeDtypeStruct((E, DA), jnp.float32),
                  jax.ShapeDtypeStruct((E, DB), jnp.float32)],
        mesh=mesh,
        compiler_params=pltpu.CompilerParams(needs_layout_passes=False),
        scratch_types=[
            pltpu.VMEM((rows_w, CH), jnp.int32),
            pltpu.VMEM((rows_w, CH), jnp.int32),
        ] + [pltpu.VMEM((CH, DA), jnp.float32)] * 4
          + [pltpu.VMEM((CH, DB), jnp.float32)] * 4
          + [pltpu.SemaphoreType.DMA] * 8,
    )
    def k(tA_h, tB_h, iA_h, iB_h, outA, outB, idxA_v, idxB_v, *rest):
        wid = lax.axis_index("s") * NC + lax.axis_index("c")
        row0 = wid * rows_w
        pltpu.sync_copy(iA_h.at[wid], idxA_v)
        pltpu.sync_copy(iB_h.at[wid], idxB_v)
        bufA = rest[0:4]
        bufB = rest[4:8]
        sA = rest[8:12]
        sB = rest[12:16]

        def issue(i, s):
            pltpu.async_copy(tA_h.at[idxA_v.at[i]], bufA[s], sA[s])
            pltpu.async_copy(tB_h.at[idxB_v.at[i]], bufB[s], sB[s])

        def finish(i, s):
            off = (row0 + i) * CH
            pltpu.make_async_copy(tA_h.at[idxA_v.at[i]], bufA[s], sA[s]).wait()
            pltpu.sync_copy(bufA[s], outA.at[pl.ds(off, CH)])
            pltpu.make_async_copy(tB_h.at[idxB_v.at[i]], bufB[s], sB[s]).wait()
            pltpu.sync_copy(bufB[s], outB.at[pl.ds(off, CH)])

        issue(0, 0)
        issue(1, 1)
        issue(2, 2)

        def body(g, _):
            for b in range(4):
                i = 4 * g + b
                @pl.when(i + 3 < rows_w)
                def _():
                    issue(i + 3, (b + 3) % 4)

                finish(i, b)
            return 0

        lax.fori_loop(0, rows_w // 4, body, 0)
        for i in range((rows_w // 4) * 4, rows_w):
            finish(i, i % 4)

    return k(tA, tB, idxA3, idxB3)


def _sc_scatter_add(m, dst3, n_rows, zslab):
    E, D = m.shape
    rows_w = E // (NW * CH)
    slab = n_rows // NS
    mesh = plsc.VectorSubcoreMesh(core_axis_name="c", subcore_axis_name="s")

    @functools.partial(
        pl.kernel,
        out_type=jax.ShapeDtypeStruct((NC, n_rows, D), jnp.float32),
        mesh=mesh,
        compiler_params=pltpu.CompilerParams(needs_layout_passes=False),
        scratch_types=[
            pltpu.VMEM((rows_w, CH), jnp.int32),
            pltpu.VMEM_SHARED((n_rows, D), jnp.float32),
        ] + [pltpu.VMEM((CH, D), jnp.float32)] * 3
          + [pltpu.SemaphoreType.DMA] * 3,
    )
    def k(m_h, dst_h, z_h, out, idx_v, acc, *bufs):
        nbuf = 3
        buf = bufs[:nbuf]
        sem = bufs[nbuf:2 * nbuf]
        cid = lax.axis_index("c")
        sid = lax.axis_index("s")
        wid = sid * NC + cid
        row0 = wid * rows_w
        pltpu.sync_copy(z_h, acc.at[pl.ds(sid * slab, slab)])
        pltpu.sync_copy(dst_h.at[wid], idx_v)
        plsc.subcore_barrier()

        def l_issue(i, s):
            pltpu.async_copy(m_h.at[pl.ds((row0 + i) * CH, CH)], buf[s], sem[s])

        def l_wait(i, s):
            pltpu.make_async_copy(
                m_h.at[pl.ds((row0 + i) * CH, CH)], buf[s], sem[s]).wait()

        l_issue(0, 0)
        l_issue(1, 1)

        def body(g, _):
            for b in range(nbuf):
                i = nbuf * g + b

                @pl.when(i + 2 < rows_w)
                def _():
                    l_issue(i + 2, (b + 2) % nbuf)

                l_wait(i, b)
                pltpu.sync_copy(buf[b], acc.at[idx_v.at[i]], add=True)
            return 0

        lax.fori_loop(0, rows_w // nbuf, body, 0)
        for i in range((rows_w // nbuf) * nbuf, rows_w):
            l_wait(i, i % nbuf)
            pltpu.sync_copy(buf[i % nbuf], acc.at[idx_v.at[i]], add=True)
        plsc.subcore_barrier()
        pltpu.sync_copy(acc.at[pl.ds(sid * slab, slab)],
                        out.at[cid, pl.ds(sid * slab, slab)])

    return k(m, dst3, zslab)


def _sc_edge_sumsq(pos4, il3, ir3):
    _, rows_w, _ = il3.shape
    E = NW * rows_w * CH
    per_w = rows_w * CH
    N4 = pos4.shape[0]
    groups = CH // 16
    mesh = plsc.VectorSubcoreMesh(core_axis_name="c", subcore_axis_name="s")

    @functools.partial(
        pl.kernel,
        out_type=jax.ShapeDtypeStruct((E,), jnp.float32),
        mesh=mesh,
        compiler_params=pltpu.CompilerParams(needs_layout_passes=False),
        scratch_types=[
            pltpu.VMEM((N4,), jnp.float32),
            pltpu.VMEM((rows_w, CH), jnp.int32),
            pltpu.VMEM((rows_w, CH), jnp.int32),
            pltpu.VMEM((per_w,), jnp.float32),
        ],
    )
    def k(pos_h, iL_h, iR_h, out, pos_v, iL_v, iR_v, out_v):
        wid = lax.axis_index("s") * NC + lax.axis_index("c")
        pltpu.sync_copy(pos_h, pos_v)
        pltpu.sync_copy(iL_h.at[wid], iL_v)
        pltpu.sync_copy(iR_h.at[wid], iR_v)

        def body(r, _):
            for g in range(groups):
                il = iL_v[r, pl.ds(g * 16, 16)] * 4
                ir = iR_v[r, pl.ds(g * 16, 16)] * 4
                acc = jnp.zeros((16,), jnp.float32)
                for c in range(3):
                    a = plsc.load_gather(pos_v, [il + c])
                    b = plsc.load_gather(pos_v, [ir + c])
                    d = a - b
                    acc = acc + d * d
                out_v[pl.ds(r * CH + g * 16, 16)] = acc
            return 0

        lax.fori_loop(0, rows_w, body, 0)
        pltpu.sync_copy(out_v, out.at[pl.ds(wid * per_w, per_w)])

    return k(pos4, il3, ir3)


def _silu(v):
    return v * jax.nn.sigmoid(v)


def _tc_embed(xg, W_emb, b_emb, WA, WB, nb):
    N, D = xg.shape
    H = W_emb.shape[1]
    grid = N // nb

    def body(x_ref, we, be, wa, wb, h_ref, a_ref, b_ref):
        h = jnp.dot(x_ref[...], we[...],
                    preferred_element_type=jnp.float32) + be[...]
        h_ref[...] = h
        a_ref[...] = jnp.dot(h, wa[...], preferred_element_type=jnp.float32)
        b_ref[...] = jnp.dot(h, wb[...], preferred_element_type=jnp.float32)

    w_spec = [pl.BlockSpec((D, H), lambda i: (0, 0)),
              pl.BlockSpec((1, H), lambda i: (0, 0)),
              pl.BlockSpec((H, H), lambda i: (0, 0)),
              pl.BlockSpec((H, H), lambda i: (0, 0))]
    return pl.pallas_call(
        body,
        grid=(grid,),
        in_specs=[pl.BlockSpec((nb, D), lambda i: (i, 0))] + w_spec,
        out_specs=[pl.BlockSpec((nb, H), lambda i: (i, 0))] * 3,
        out_shape=[jax.ShapeDtypeStruct((N, H), jnp.float32)] * 3,
    )(xg, W_emb, b_emb, WA, WB)


def _tc_message(Asrc, Bdst, sumsq_c, w_inv, bm1, Wm2, bm2, eb):
    E, H = Asrc.shape
    grid = E // eb
    rb = eb // 128

    def body(a_ref, b_ref, s_ref, wi, b1, w2, b2, m_ref):
        invc = jnp.sqrt(s_ref[...] + 1e-12)
        row = lax.broadcasted_iota(jnp.int32, (eb, rb), 0) // 128
        col = lax.broadcasted_iota(jnp.int32, (eb, rb), 1)
        sel = (row == col).astype(jnp.float32)
        t = jnp.dot(sel, invc, preferred_element_type=jnp.float32)
        lane = lax.broadcasted_iota(jnp.int32, (eb, 128), 1)
        rmod = lax.broadcasted_iota(jnp.int32, (eb, 128), 0) % 128
        inv_col = jnp.sum(jnp.where(lane == rmod, t, 0.0), axis=1,
                          keepdims=True)
        z = a_ref[...] + b_ref[...] + inv_col * wi[...] + b1[...]
        z = _silu(z)
        mm = jnp.dot(z, w2[...], preferred_element_type=jnp.float32) + b2[...]
        m_ref[...] = _silu(mm)

    return pl.pallas_call(
        body,
        grid=(grid,),
        in_specs=[
            pl.BlockSpec((eb, H), lambda i: (i, 0)),
            pl.BlockSpec((eb, H), lambda i: (i, 0)),
            pl.BlockSpec((rb, 128), lambda i: (i, 0)),
            pl.BlockSpec((1, H), lambda i: (0, 0)),
            pl.BlockSpec((1, H), lambda i: (0, 0)),
            pl.BlockSpec((H, H), lambda i: (0, 0)),
            pl.BlockSpec((1, H), lambda i: (0, 0)),
        ],
        out_specs=pl.BlockSpec((eb, H), lambda i: (i, 0)),
        out_shape=jax.ShapeDtypeStruct((E, H), jnp.float32),
    )(Asrc, Bdst, sumsq_c, w_inv, bm1, Wm2, bm2)


def _tc_update(h, agg0, agg1, W1h, W1a, b1, W2, b2, WA, WB, nb):
    N, H = h.shape
    grid = N // nb
    n_out = 3 if WA is not None else 1

    def body(h_ref, a0_ref, a1_ref, w1h, w1a, b1r, w2, b2r, *outs):
        h_blk = h_ref[...]
        agg = a0_ref[...] + a1_ref[...]
        u = jnp.dot(h_blk, w1h[...], preferred_element_type=jnp.float32)
        u += jnp.dot(agg, w1a[...], preferred_element_type=jnp.float32)
        u = _silu(u + b1r[...])
        u = jnp.dot(u, w2[...], preferred_element_type=jnp.float32) + b2r[...]
        hn = h_blk + u
        if n_out == 3:
            wa, wb, hn_ref, a_ref, b_ref = outs
            hn_ref[...] = hn
            a_ref[...] = jnp.dot(hn, wa[...],
                                 preferred_element_type=jnp.float32)
            b_ref[...] = jnp.dot(hn, wb[...],
                                 preferred_element_type=jnp.float32)
        else:
            (hn_ref,) = outs
            hn_ref[...] = hn

    in_specs = [
        pl.BlockSpec((nb, H), lambda i: (i, 0)),
        pl.BlockSpec((nb, H), lambda i: (i, 0)),
        pl.BlockSpec((nb, H), lambda i: (i, 0)),
        pl.BlockSpec((H, H), lambda i: (0, 0)),
        pl.BlockSpec((H, H), lambda i: (0, 0)),
        pl.BlockSpec((1, H), lambda i: (0, 0)),
        pl.BlockSpec((H, H), lambda i: (0, 0)),
        pl.BlockSpec((1, H), lambda i: (0, 0)),
    ]
    args = [h, agg0, agg1, W1h, W1a, b1, W2, b2]
    if n_out == 3:
        in_specs += [pl.BlockSpec((H, H), lambda i: (0, 0))] * 2
        args += [WA, WB]
    return pl.pallas_call(
        body,
        grid=(grid,),
        in_specs=in_specs,
        out_specs=[pl.BlockSpec((nb, H), lambda i: (i, 0))] * n_out,
        out_shape=[jax.ShapeDtypeStruct((N, H), jnp.float32)] * n_out,
    )(*args)


def _tc_pool(h, xb_row, Wp1, bp1, Wp2, bp2, Wq1, bq1, Wq2, bq2, n_graphs, nb):
    N, H = h.shape
    grid = N // nb

    def body(h_ref, xb_ref, wp1, bp1r, wp2, bp2r, wq1, bq1r, wq2, bq2r,
             out_ref, acc):
        step = pl.program_id(0)

        @pl.when(step == 0)
        def _():
            acc[...] = jnp.zeros_like(acc)

        hp = _silu(jnp.dot(h_ref[...], wp1[...],
                           preferred_element_type=jnp.float32) + bp1r[...])
        hp = jnp.dot(hp, wp2[...], preferred_element_type=jnp.float32) \
            + bp2r[...]
        rows = lax.broadcasted_iota(jnp.int32, (n_graphs, nb), 0)
        mask = (rows == xb_ref[0]).astype(jnp.float32)
        acc[...] += jnp.dot(mask, hp, preferred_element_type=jnp.float32)

        @pl.when(step == grid - 1)
        def _():
            p = acc[...]
            q = _silu(jnp.dot(p, wq1[...],
                              preferred_element_type=jnp.float32) + bq1r[...])
            out_ref[...] = jnp.dot(q, wq2[...],
                                   preferred_element_type=jnp.float32) \
                + bq2r[...]

    return pl.pallas_call(
        body,
        grid=(grid,),
        in_specs=[
            pl.BlockSpec((nb, H), lambda i: (i, 0)),
            pl.BlockSpec((1, 1, nb), lambda i: (i, 0, 0)),
            pl.BlockSpec((H, H), lambda i: (0, 0)),
            pl.BlockSpec((1, H), lambda i: (0, 0)),
            pl.BlockSpec((H, H), lambda i: (0, 0)),
            pl.BlockSpec((1, H), lambda i: (0, 0)),
            pl.BlockSpec((H, H), lambda i: (0, 0)),
            pl.BlockSpec((1, H), lambda i: (0, 0)),
            pl.BlockSpec((H, 1), lambda i: (0, 0)),
            pl.BlockSpec((1, 1), lambda i: (0, 0)),
        ],
        out_specs=pl.BlockSpec((n_graphs, 1), lambda i: (0, 0)),
        out_shape=jax.ShapeDtypeStruct((n_graphs, 1), jnp.float32),
        scratch_shapes=[pltpu.VMEM((n_graphs, H), jnp.float32)],
    )(h, xb_row, Wp1, bp1, Wp2, bp2, Wq1, bq1, Wq2, bq2)


def kernel(x, pos, feat_ind, adj, inv_ind, x_batch, W_emb, b_emb, Wm1, bm1,
           Wm2, bm2, Wu1, bu1, Wu2, bu2, Wpre1, bpre1, Wpre2, bpre2, Wpost1,
           bpost1, Wpost2, bpost2):
    N, D = x.shape
    H = W_emb.shape[1]
    L = Wm1.shape[0]
    E = adj.shape[1]
    n_graphs = 64

    xg = jnp.take(x, feat_ind, axis=0)
    NP = 10112
    EP = 327680
    pad = EP - E
    rows_w = EP // (NW * CH)
    src3 = jnp.concatenate([adj[0], jnp.zeros((pad,), jnp.int32)]
                           ).reshape(NW, rows_w, CH)
    dst3 = jnp.concatenate([adj[1], jnp.full((pad,), N, jnp.int32)]
                           ).reshape(NW, rows_w, CH)
    il3 = jnp.concatenate([inv_ind[0], jnp.zeros((pad,), jnp.int32)]
                          ).reshape(NW, rows_w, CH)
    ir3 = jnp.concatenate([inv_ind[1], jnp.zeros((pad,), jnp.int32)]
                          ).reshape(NW, rows_w, CH)
    pos4 = jnp.concatenate([pos, jnp.zeros((N, 1), jnp.float32)],
                           axis=1).reshape(-1)
    zslab = jnp.zeros((NP // NS, H), jnp.float32)

    h, A, Bt = _tc_embed(xg, W_emb, b_emb[None], Wm1[0, :H], Wm1[0, H:2 * H],
                         nb=2000)

    sumsq = _sc_edge_sumsq(pos4, il3, ir3).reshape(EP // 128, 128)

    for l in range(L):
        Asrc, Bdst = _sc_gather2(A, Bt, src3, dst3)
        m = _tc_message(Asrc, Bdst, sumsq, Wm1[l, 2 * H:2 * H + 1, :],
                        bm1[l][None], Wm2[l], bm2[l][None], eb=2048)
        aggp = _sc_scatter_add(m, dst3, NP, zslab)
        if l < L - 1:
            h, A, Bt = _tc_update(h, aggp[0], aggp[1], Wu1[l, :H],
                                  Wu1[l, H:], bu1[l][None], Wu2[l],
                                  bu2[l][None], Wm1[l + 1, :H],
                                  Wm1[l + 1, H:2 * H], nb=2000)
        else:
            (h,) = _tc_update(h, aggp[0], aggp[1], Wu1[l, :H], Wu1[l, H:],
                              bu1[l][None], Wu2[l], bu2[l][None],
                              None, None, nb=2000)

    xb3 = x_batch.astype(jnp.int32).reshape(N // 2000, 1, 2000)
    out = _tc_pool(h, xb3, Wpre1, bpre1[None],
                   Wpre2, bpre2[None], Wpost1, bpost1[None], Wpost2,
                   bpost2[None, :], n_graphs, nb=2000)
    return jnp.squeeze(out)
